# Initial kernel scaffold; baseline (speedup 1.0000x reference)
import jax
import jax.numpy as jnp
from jax.experimental import pallas as pl


def kernel(x, edge_index, batch, params):
    raise NotImplementedError("write your pallas kernel here")


# speedup vs baseline: 10.8535x; 10.8535x over previous
import functools

import jax
import jax.numpy as jnp
from jax import lax
from jax.experimental import pallas as pl
from jax.experimental.pallas import tpu as pltpu
from jax.experimental.pallas import tpu_sc as plsc

_N = 10000
_E = 320000
_NG = 512
_NS = 16
_K = 80
_ROWS_PER_TILE = _E // _NS // _K
_NB = 25
_IB = _ROWS_PER_TILE // _NB
_EB = 5

_f32 = jnp.float32
_HIGH = lax.Precision.HIGHEST


def _mesh():
    return plsc.VectorSubcoreMesh(core_axis_name="c", subcore_axis_name="s")


_NBLK = _E // (8 * _K)


def _sc_deg(dst3d, ones_h, zerosN_h):
    @functools.partial(
        pl.kernel,
        out_type=(jax.ShapeDtypeStruct((_N,), _f32),) * 2,
        mesh=_mesh(),
        scratch_types=[
            pltpu.VMEM((8, _K), jnp.int32),
            pltpu.VMEM((_K,), _f32),
            pltpu.VMEM_SHARED((_N,), _f32),
        ],
    )
    def k(dst_h, ones_hbm, zeros_hbm, deg0, deg1, didx, ones_v, dacc):
        c = lax.axis_index("c")
        s = lax.axis_index("s")
        wid = c * _NS + s
        pltpu.sync_copy(ones_hbm, ones_v)

        @pl.when(s == 0)
        def _():
            pltpu.sync_copy(zeros_hbm, dacc)

        plsc.subcore_barrier()

        def step(t, carry):
            blk = wid + 32 * t

            @pl.when(blk < _NBLK)
            def _():
                pltpu.sync_copy(dst_h.at[blk], didx)
                for j in range(8):
                    pltpu.sync_copy(ones_v, dacc.at[didx.at[j]], add=True)

            return carry

        lax.fori_loop(0, 16, step, 0)
        plsc.subcore_barrier()

        @pl.when(jnp.logical_and(c == 0, s == 0))
        def _():
            pltpu.sync_copy(dacc, deg0)

        @pl.when(jnp.logical_and(c == 1, s == 0))
        def _():
            pltpu.sync_copy(dacc, deg1)

    return k(dst3d, ones_h, zerosN_h)


def _sc_agg(ha, hb, src3d, dst3d, zeros_h, w):
    @functools.partial(
        pl.kernel,
        out_type=(jax.ShapeDtypeStruct((_N, w), _f32),) * 2,
        mesh=_mesh(),
        scratch_types=[
            pltpu.VMEM((8, _K), jnp.int32),
            pltpu.VMEM((8, _K), jnp.int32),
            pltpu.VMEM((_K, w), _f32),
            pltpu.VMEM((125, w), _f32),
            pltpu.VMEM_SHARED((_N, w), _f32),
            pltpu.SemaphoreType.DMA,
        ],
    )
    def k(ha_h, hb_h, src_h, dst_h, zeros_hbm, outa, outb,
          sidx, didx, rows, zv, acc, semg):
        c = lax.axis_index("c")
        s = lax.axis_index("s")
        pltpu.sync_copy(zeros_hbm, zv)

        @pl.when(s < 10)
        def _():
            for j in range(8):
                pltpu.sync_copy(zv, acc.at[pl.ds(1000 * s + 125 * j, 125)])

        plsc.subcore_barrier()

        def step(t, carry):
            blk = s + _NS * t

            @pl.when(blk < _NBLK)
            def _():
                pltpu.sync_copy(src_h.at[blk], sidx)
                pltpu.sync_copy(dst_h.at[blk], didx)
                for j in range(8):
                    @pl.when(c == 0)
                    def _():
                        pltpu.async_copy(ha_h.at[sidx.at[j]], rows, semg).wait()

                    @pl.when(c == 1)
                    def _():
                        pltpu.async_copy(hb_h.at[sidx.at[j]], rows, semg).wait()

                    pltpu.sync_copy(rows, acc.at[didx.at[j]], add=True)

            return carry

        lax.fori_loop(0, (_NBLK + _NS - 1) // _NS, step, 0)
        plsc.subcore_barrier()

        @pl.when(jnp.logical_and(c == 0, s < 10))
        def _():
            pltpu.sync_copy(acc.at[pl.ds(1000 * s, 1000)],
                            outa.at[pl.ds(1000 * s, 1000)])

        @pl.when(jnp.logical_and(c == 1, s < 10))
        def _():
            pltpu.sync_copy(acc.at[pl.ds(1000 * s, 1000)],
                            outb.at[pl.ds(1000 * s, 1000)])

    return k(ha, hb, src3d, dst3d, zeros_h)


def _sc_agg_edges(h, src3d, dst3d, zeros_h):
    @functools.partial(
        pl.kernel,
        out_type=(jax.ShapeDtypeStruct((_N, 128), _f32),) * 2,
        mesh=_mesh(),
        scratch_types=[
            pltpu.VMEM((8, _K), jnp.int32),
            pltpu.VMEM((8, _K), jnp.int32),
            pltpu.VMEM((_K, 128), _f32),
            pltpu.VMEM((125, 128), _f32),
            pltpu.VMEM_SHARED((_N, 128), _f32),
            pltpu.SemaphoreType.DMA,
        ],
    )
    def k(h_h, src_h, dst_h, zeros_hbm, out0, out1,
          sidx, didx, rows, zv, acc, semg):
        c = lax.axis_index("c")
        s = lax.axis_index("s")
        wid = c * _NS + s
        pltpu.sync_copy(zeros_hbm, zv)

        @pl.when(s < 10)
        def _():
            for j in range(8):
                pltpu.sync_copy(zv, acc.at[pl.ds(1000 * s + 125 * j, 125)])

        plsc.subcore_barrier()

        def step(t, carry):
            blk = wid + 32 * t

            @pl.when(blk < _NBLK)
            def _():
                pltpu.sync_copy(src_h.at[blk], sidx)
                pltpu.sync_copy(dst_h.at[blk], didx)
                for j in range(8):
                    pltpu.async_copy(h_h.at[sidx.at[j]], rows, semg).wait()
                    pltpu.sync_copy(rows, acc.at[didx.at[j]], add=True)

            return carry

        lax.fori_loop(0, 16, step, 0)
        plsc.subcore_barrier()

        @pl.when(jnp.logical_and(c == 0, s < 10))
        def _():
            pltpu.sync_copy(acc.at[pl.ds(1000 * s, 1000)],
                            out0.at[pl.ds(1000 * s, 1000)])

        @pl.when(jnp.logical_and(c == 1, s < 10))
        def _():
            pltpu.sync_copy(acc.at[pl.ds(1000 * s, 1000)],
                            out1.at[pl.ds(1000 * s, 1000)])

    return k(h, src3d, dst3d, zeros_h)


def _sc_agg_l0(xp, dp, src3d, dst3d, zeros2d, zerosN_h):
    @functools.partial(
        pl.kernel,
        out_type=(jax.ShapeDtypeStruct((_N, 128), _f32),
                  jax.ShapeDtypeStruct((_N, 128), _f32),
                  jax.ShapeDtypeStruct((_N,), _f32),
                  jax.ShapeDtypeStruct((_N,), _f32)),
        mesh=_mesh(),
        scratch_types=[
            pltpu.VMEM((8, _K), jnp.int32),
            pltpu.VMEM((8, _K), jnp.int32),
            pltpu.VMEM((_K, 128), _f32),
            pltpu.VMEM((_K,), _f32),
            pltpu.VMEM((125, 128), _f32),
            pltpu.VMEM_SHARED((_N, 128), _f32),
            pltpu.VMEM_SHARED((_N,), _f32),
            pltpu.SemaphoreType.DMA,
        ],
    )
    def k(xp_h, dp_h, src_h, dst_h, z2_h, zN_h,
          s0, s1, sd0, sd1,
          sidx, didx, rows, vals, zv, acc, accd, semg):
        c = lax.axis_index("c")
        s = lax.axis_index("s")
        wid = c * _NS + s
        pltpu.sync_copy(z2_h, zv)

        @pl.when(s < 10)
        def _():
            for j in range(8):
                pltpu.sync_copy(zv, acc.at[pl.ds(1000 * s + 125 * j, 125)])

        @pl.when(s == 15)
        def _():
            pltpu.sync_copy(zN_h, accd)

        plsc.subcore_barrier()

        def step(t, carry):
            blk = wid + 32 * t

            @pl.when(blk < _NBLK)
            def _():
                pltpu.sync_copy(src_h.at[blk], sidx)
                pltpu.sync_copy(dst_h.at[blk], didx)
                for j in range(8):
                    pltpu.async_copy(xp_h.at[sidx.at[j]], rows, semg).wait()
                    pltpu.sync_copy(rows, acc.at[didx.at[j]], add=True)
                    pltpu.async_copy(dp_h.at[sidx.at[j]], vals, semg).wait()
                    pltpu.sync_copy(vals, accd.at[didx.at[j]], add=True)

            return carry

        lax.fori_loop(0, 16, step, 0)
        plsc.subcore_barrier()

        @pl.when(jnp.logical_and(c == 0, s < 10))
        def _():
            pltpu.sync_copy(acc.at[pl.ds(1000 * s, 1000)],
                            s0.at[pl.ds(1000 * s, 1000)])

        @pl.when(jnp.logical_and(c == 1, s < 10))
        def _():
            pltpu.sync_copy(acc.at[pl.ds(1000 * s, 1000)],
                            s1.at[pl.ds(1000 * s, 1000)])

        @pl.when(jnp.logical_and(c == 0, s == 15))
        def _():
            pltpu.sync_copy(accd, sd0)

        @pl.when(jnp.logical_and(c == 1, s == 15))
        def _():
            pltpu.sync_copy(accd, sd1)

    return k(xp, dp, src3d, dst3d, zeros2d, zerosN_h)


def _sc_pool(emb, batch3d, zeros_h):
    @functools.partial(
        pl.kernel,
        out_type=jax.ShapeDtypeStruct((_NG, 128), _f32),
        mesh=_mesh(),
        scratch_types=[
            pltpu.VMEM((200, 128), _f32),
            pltpu.VMEM((2, 100), jnp.int32),
            pltpu.VMEM((32, 128), _f32),
            pltpu.VMEM_SHARED((_NG, 128), _f32),
        ],
    )
    def k(emb_h, batch_h, zeros_hbm, g, rows, bidx, zv, gacc):
        c = lax.axis_index("c")
        s = lax.axis_index("s")

        @pl.when(c == 0)
        def _():
            pltpu.sync_copy(zeros_hbm, zv)
            pltpu.sync_copy(zv, gacc.at[pl.ds(32 * s, 32)])
            plsc.subcore_barrier()
            for j in range(4):
                chunk = s + 16 * j

                @pl.when(chunk < 50)
                def _():
                    pltpu.sync_copy(emb_h.at[pl.ds(chunk * 200, 200)], rows)
                    pltpu.sync_copy(batch_h.at[chunk], bidx)
                    for t in range(2):
                        pltpu.sync_copy(rows.at[pl.ds(t * 100, 100)],
                                        gacc.at[bidx.at[t]], add=True)

            plsc.subcore_barrier()
            pltpu.sync_copy(gacc.at[pl.ds(32 * s, 32)], g.at[pl.ds(32 * s, 32)])

    return k(emb, batch3d, zeros_h)


_R = 1000
_GRID = _N // _R


def _rspec(width):
    return pl.BlockSpec((_R, width), lambda i: (i, 0))


def _fspec(shape):
    return pl.BlockSpec(shape, lambda i: (0, 0))


def _tc_prologue(x, deg0, deg1):
    def body(x_r, d0_r, d1_r, xp_r, dp_r, dis_r):
        deg = d0_r[...] + d1_r[...]
        dis = 1.0 / jnp.sqrt(deg + 1.0)
        xp_r[...] = x_r[...] * dis
        dp_r[...] = deg * dis
        dis_r[...] = dis

    return pl.pallas_call(
        body,
        grid=(_GRID,),
        in_specs=[_rspec(128), _rspec(1), _rspec(1)],
        out_specs=(_rspec(128), _rspec(1), _rspec(1)),
        out_shape=(jax.ShapeDtypeStruct((_N, 128), _f32),
                   jax.ShapeDtypeStruct((_N, 1), _f32),
                   jax.ShapeDtypeStruct((_N, 1), _f32)),
    )(x, deg0, deg1)


import math
_BN_SCALE = 1.0 / math.sqrt(1.0 + 1e-5)


def _tc_layer0(s0, s1, xp, sd0, sd1, dp, dis,
               w0x, w0d, b0, g0, be0, w1):
    def body(s0_r, s1_r, xp_r, sd0_r, sd1_r, dp_r, dis_r,
             w0x_r, w0d_r, b0_r, g0_r, be0_r, w1_r, ua_r, ub_r):
        dis = dis_r[...]
        aggx = (s0_r[...] + s1_r[...] + xp_r[...]) * dis
        aggd = (sd0_r[...] + sd1_r[...] + dp_r[...]) * dis
        h = (jnp.dot(aggx, w0x_r[...], precision=_HIGH,
                     preferred_element_type=_f32)
             + aggd * w0d_r[...] + b0_r[...])
        h = h * _BN_SCALE * g0_r[...] + be0_r[...]
        h = jnp.maximum(h, 0.0)
        u = jnp.dot(h, w1_r[...], precision=_HIGH,
                    preferred_element_type=_f32) * dis
        ua_r[...] = u[:, :128]
        ub_r[...] = u[:, 128:]

    return pl.pallas_call(
        body,
        grid=(_GRID,),
        in_specs=[_rspec(128), _rspec(128), _rspec(128),
                  _rspec(1), _rspec(1), _rspec(1), _rspec(1),
                  _fspec((128, 256)), _fspec((1, 256)), _fspec((1, 256)),
                  _fspec((1, 256)), _fspec((1, 256)), _fspec((256, 256))],
        out_specs=(_rspec(128), _rspec(128)),
        out_shape=(jax.ShapeDtypeStruct((_N, 128), _f32),
                   jax.ShapeDtypeStruct((_N, 128), _f32)),
    )(s0, s1, xp, sd0, sd1, dp, dis, w0x, w0d, b0, g0, be0, w1)


def _tc_mid(sa, sb, ua, ub, dis, gbn, bbn, bias, w, din, dout, split_out):
    half_in = din // 2
    half_out = dout // 2

    def body(sa_r, sb_r, ua_r, ub_r, dis_r, g_r, b_r, bias_r, w_r, *outs):
        dis = dis_r[...]
        h = jnp.concatenate(
            [sa_r[...] + ua_r[...], sb_r[...] + ub_r[...]], axis=1) * dis
        h = h + bias_r[...]
        h = h * _BN_SCALE * g_r[...] + b_r[...]
        h = jnp.maximum(h, 0.0)
        v = jnp.dot(h, w_r[...], precision=_HIGH,
                    preferred_element_type=_f32) * dis
        if split_out:
            outs[0][...] = v[:, :half_out]
            outs[1][...] = v[:, half_out:]
        else:
            outs[0][...] = v

    if split_out:
        out_specs = (_rspec(half_out), _rspec(half_out))
        out_shape = (jax.ShapeDtypeStruct((_N, half_out), _f32),
                     jax.ShapeDtypeStruct((_N, half_out), _f32))
    else:
        out_specs = _rspec(dout)
        out_shape = jax.ShapeDtypeStruct((_N, dout), _f32)
    return pl.pallas_call(
        body,
        grid=(_GRID,),
        in_specs=[_rspec(half_in), _rspec(half_in),
                  _rspec(half_in), _rspec(half_in), _rspec(1),
                  _fspec((1, din)), _fspec((1, din)), _fspec((1, din)),
                  _fspec((din, dout))],
        out_specs=out_specs,
        out_shape=out_shape,
    )(sa, sb, ua, ub, dis, gbn, bbn, bias, w)


def _tc_emb(s0, s1, u3, dis, g3, b3, bias3):
    def body(s0_r, s1_r, u3_r, dis_r, g_r, b_r, bias_r, emb_r):
        h = (s0_r[...] + s1_r[...] + u3_r[...]) * dis_r[...]
        h = h + bias_r[...]
        h = h * _BN_SCALE * g_r[...] + b_r[...]
        emb_r[...] = jnp.where(h > 0, h, 0.2 * h)

    return pl.pallas_call(
        body,
        grid=(_GRID,),
        in_specs=[_rspec(128), _rspec(128), _rspec(128), _rspec(1),
                  _fspec((1, 128)), _fspec((1, 128)), _fspec((1, 128))],
        out_specs=_rspec(128),
        out_shape=jax.ShapeDtypeStruct((_N, 128), _f32),
    )(s0, s1, u3, dis, g3, b3, bias3)


def _ln(x, g, b):
    m = jnp.mean(x, axis=1, keepdims=True)
    v = jnp.mean((x - m) ** 2, axis=1, keepdims=True)
    return (x - m) / jnp.sqrt(v + 1e-5) * g + b


def _gelu(x):
    return 0.5 * x * (1.0 + lax.erf(x * (1.0 / jnp.sqrt(jnp.float32(2.0)))))


def _dotT(a, b):
    return lax.dot_general(a, b, (((1,), (1,)), ((), ())),
                           precision=_HIGH, preferred_element_type=_f32)


def _mm(a, b):
    return jnp.dot(a, b, precision=_HIGH, preferred_element_type=_f32)


def _tc_popgraph(g, pp):
    def body(g_r,
             l1w, l1b, lng, lnb, l2w, l2b, temp, theta, mu, sigma,
             inw, inb, inlng, inlnb, g0w, g0b, g1w, g1b,
             c1w, c1b, c1g, c1bb, c2w, c2b, c2g, c2bb, c3w, c3b,
             logits_r, kl_r):
        gg = g_r[...]
        h = _mm(gg, l1w[...]) + l1b[...]
        h = _ln(h, lng[...], lnb[...])
        h = _gelu(h)
        latv = _mm(h, l2w[...]) + l2b[...]

        latsq = latv * latv
        n2c = jnp.sum(latsq, axis=1, keepdims=True)
        ones_r = jnp.ones((1, 64), _f32)
        n2r = lax.dot_general(ones_r, latsq, (((1,), (1,)), ((), ())),
                              precision=_HIGH, preferred_element_type=_f32)
        gram = _dotT(latv, latv)
        d2 = jnp.maximum(n2c + n2r - 2.0 * gram, 0.0)
        dist = jnp.sqrt(d2 + 1e-6)
        adj = jax.nn.sigmoid(-temp[0, 0] * dist + theta[0, 0])
        ri = lax.broadcasted_iota(jnp.int32, (_NG, _NG), 0)
        ci = lax.broadcasted_iota(jnp.int32, (_NG, _NG), 1)
        eyef = (ri == ci).astype(_f32)
        adj = adj * (1.0 - eyef) + eyef

        mask = (adj > 0.5).astype(_f32)
        A = adj * mask
        d_bar = jnp.sum(A, axis=1, keepdims=True)
        cgrid = ci.astype(_f32)
        delta = d_bar - cgrid
        S = jnp.exp(-delta * delta)
        numer = jnp.sum(S, axis=0, keepdims=True)
        q = numer / (jnp.sum(numer) + 1e-8)
        crow = lax.broadcasted_iota(jnp.int32, (1, _NG), 1).astype(_f32)
        r = jnp.exp(-(crow - mu[0, 0]) ** 2 / (2.0 * sigma[0, 0] ** 2))
        r = r / (jnp.sum(r) + 1e-8)
        kl = jnp.sum(q * jnp.log(q / (r + 1e-8) + 1e-8))
        kl = jnp.clip(kl, 0.0, 10.0)
        kl_r[...] = jnp.reshape(kl, (1, 1))

        dispc = jnp.where(d_bar > 0, 1.0 / jnp.sqrt(d_bar), 0.0)
        dispr = lax.dot_general(dispc, eyef, (((0,), (0,)), ((), ())),
                                precision=_HIGH, preferred_element_type=_f32)
        An = dispc * A * dispr

        h = _mm(gg, inw[...]) + inb[...]
        h = _ln(h, inlng[...], inlnb[...])
        h = _gelu(h)
        h = jnp.maximum(_mm(An, _mm(h, g0w[...])) + g0b[...], 0.0)
        h = jnp.maximum(_mm(An, _mm(h, g1w[...])) + g1b[...], 0.0)
        h = _mm(h, c1w[...]) + c1b[...]
        h = _ln(h, c1g[...], c1bb[...])
        h = _gelu(h)
        h = _mm(h, c2w[...]) + c2b[...]
        h = _ln(h, c2g[...], c2bb[...])
        h = _gelu(h)
        logits_r[...] = _mm(h, c3w[...]) + c3b[...]

    n = _NG
    specs = [pl.BlockSpec(a.shape, lambda i: tuple(0 for _ in a.shape))
             for a in pp]
    return pl.pallas_call(
        body,
        grid=(1,),
        in_specs=[pl.BlockSpec((n, 128), lambda i: (0, 0))] + specs,
        out_specs=(pl.BlockSpec((n, 10), lambda i: (0, 0)),
                   pl.BlockSpec((1, 1), lambda i: (0, 0))),
        out_shape=(jax.ShapeDtypeStruct((n, 10), _f32),
                   jax.ShapeDtypeStruct((1, 1), _f32)),
    )(g, *pp)


def kernel(x, edge_index, batch, params):
    p = params
    src3d = edge_index[0].reshape(_NBLK, 8, _K)
    dst3d = edge_index[1].reshape(_NBLK, 8, _K)
    batch3d = batch.reshape(50, 2, 100)

    zerosN = jnp.zeros((_N,), _f32)
    ones80 = jnp.ones((_K,), _f32)
    z128 = jnp.zeros((125, 128), _f32)
    z32 = jnp.zeros((32, 128), _f32)

    def col(v):
        return v.reshape(_N, 1)

    deg0, deg1 = _sc_deg(dst3d, ones80, zerosN)
    xp, dp, dis = _tc_prologue(x, col(deg0), col(deg1))

    s0, s1, sd0, sd1 = _sc_agg_l0(xp, dp.reshape(_N), src3d, dst3d,
                                  z128, zerosN)

    def row(v, width):
        return v.reshape(1, width)

    u1a, u1b = _tc_layer0(
        s0, s1, xp, col(sd0), col(sd1), dp, dis,
        p["f1_w0"][:128], p["f1_w0"][128:129], row(p["f1_b0"], 256),
        row(p["f1_bn_g0"], 256), row(p["f1_bn_b0"], 256), p["f1_w1"])

    s1a, s1b = _sc_agg(u1a, u1b, src3d, dst3d, z128, 128)
    u2a, u2b = _tc_mid(s1a, s1b, u1a, u1b, dis,
                       row(p["f1_bn_g1"], 256), row(p["f1_bn_b1"], 256),
                       row(p["f1_b1"], 256), p["f1_w2"], 256, 256, True)

    s2a, s2b = _sc_agg(u2a, u2b, src3d, dst3d, z128, 128)
    u3 = _tc_mid(s2a, s2b, u2a, u2b, dis,
                 row(p["f1_bn_g2"], 256), row(p["f1_bn_b2"], 256),
                 row(p["f1_b2"], 256), p["f1_w3"], 256, 128, False)

    s3p0, s3p1 = _sc_agg_edges(u3, src3d, dst3d, z128)
    emb = _tc_emb(s3p0, s3p1, u3, dis,
                  row(p["f1_bn_g3"], 128), row(p["f1_bn_b3"], 128),
                  row(p["f1_b3"], 128))

    g = _sc_pool(emb, batch3d, z32)

    def s11(v):
        return v.reshape(1, 1)

    pp = (p["f2_l1_w"], row(p["f2_l1_b"], 64),
          row(p["f2_ln_g"], 64), row(p["f2_ln_b"], 64),
          p["f2_l2_w"], row(p["f2_l2_b"], 64),
          s11(p["f2_temp"]), s11(p["f2_theta"]),
          s11(p["f2_mu"]), s11(p["f2_sigma"]),
          p["f3_in_w"], row(p["f3_in_b"], 256),
          row(p["f3_in_ln_g"], 256), row(p["f3_in_ln_b"], 256),
          p["f3_g0_w"], row(p["f3_g0_b"], 256),
          p["f3_g1_w"], row(p["f3_g1_b"], 256),
          p["f3_c1_w"], row(p["f3_c1_b"], 512),
          row(p["f3_c1_ln_g"], 512), row(p["f3_c1_ln_b"], 512),
          p["f3_c2_w"], row(p["f3_c2_b"], 512),
          row(p["f3_c2_ln_g"], 512), row(p["f3_c2_ln_b"], 512),
          p["f3_c3_w"], row(p["f3_c3_b"], 10))

    logits, kl = _tc_popgraph(g, pp)
    return logits, jnp.reshape(kl, ())


# speedup vs baseline: 17.9708x; 1.6558x over previous
import functools
import math

import jax
import jax.numpy as jnp
from jax import lax
from jax.experimental import pallas as pl
from jax.experimental.pallas import tpu as pltpu
from jax.experimental.pallas import tpu_sc as plsc

_N = 10000
_E = 320000
_NG = 512
_NS = 16
_K = 128

_f32 = jnp.float32
_HIGH = lax.Precision.HIGHEST


def _mesh():
    return plsc.VectorSubcoreMesh(core_axis_name="c", subcore_axis_name="s")


_NBLK = 313
_EPAD = _NBLK * 8 * _K
_NPAD = _N + 8


def _emit_pipeline(h_h, src_h, dst_h, sidx, didx, rows, acc,
                   semg, sems, semi, base0, stride, nt, elem=None):
    dummy = h_h.at[pl.ds(0, _K)]
    if elem is not None:
        dp_h, vals, accd, semg2, sems2 = elem
        dummy2 = dp_h.at[pl.ds(0, _K)]

    pltpu.sync_copy(src_h.at[base0], sidx.at[0])
    pltpu.sync_copy(dst_h.at[base0], didx.at[0])
    pltpu.async_copy(h_h.at[sidx.at[0, 0]], rows.at[0], semg)
    if elem is not None:
        pltpu.async_copy(dp_h.at[sidx.at[0, 0]], vals.at[0], semg2)

    def step(t, carry):
        blk = base0 + stride * t
        nxt = blk + stride
        slot = lax.rem(t, 2)
        nslot = lax.rem(t + 1, 2)

        @pl.when(blk < _NBLK)
        def _():
            @pl.when(nxt < _NBLK)
            def _():
                pltpu.async_copy(src_h.at[nxt], sidx.at[nslot], semi)
                pltpu.async_copy(dst_h.at[nxt], didx.at[nslot], semi)

            for j in range(8):
                p = j % 2
                q = 1 - p
                pltpu.make_async_copy(dummy, rows.at[p], semg).wait()
                pltpu.async_copy(rows.at[p], acc.at[didx.at[slot, j]],
                                 sems, add=True)
                if elem is not None:
                    pltpu.make_async_copy(dummy2, vals.at[p], semg2).wait()
                    pltpu.async_copy(vals.at[p], accd.at[didx.at[slot, j]],
                                     sems2, add=True)

                def _wait_prev():
                    pltpu.make_async_copy(dummy, rows.at[q], sems).wait()
                    if elem is not None:
                        pltpu.make_async_copy(dummy2, vals.at[q],
                                              sems2).wait()

                if j == 0:
                    @pl.when(t > 0)
                    def _():
                        _wait_prev()
                else:
                    _wait_prev()

                if j < 7:
                    pltpu.async_copy(h_h.at[sidx.at[slot, j + 1]],
                                     rows.at[q], semg)
                    if elem is not None:
                        pltpu.async_copy(dp_h.at[sidx.at[slot, j + 1]],
                                         vals.at[q], semg2)
                else:
                    @pl.when(nxt < _NBLK)
                    def _():
                        pltpu.make_async_copy(src_h.at[0], sidx.at[nslot],
                                              semi).wait()
                        pltpu.make_async_copy(dst_h.at[0], didx.at[nslot],
                                              semi).wait()
                        pltpu.async_copy(h_h.at[sidx.at[nslot, 0]],
                                         rows.at[q], semg)
                        if elem is not None:
                            pltpu.async_copy(dp_h.at[sidx.at[nslot, 0]],
                                             vals.at[q], semg2)

        return carry

    lax.fori_loop(0, nt, step, 0)
    pltpu.make_async_copy(dummy, rows.at[1], sems).wait()
    if elem is not None:
        pltpu.make_async_copy(dummy2, vals.at[1], sems2).wait()


def _zero_acc(zeros_hbm, acc, s):
    @pl.when(s < 10)
    def _():
        for j in range(8):
            pltpu.sync_copy(zeros_hbm, acc.at[pl.ds(1000 * s + 125 * j, 125)])


def _readout(acc, out0, out1, c, s):
    @pl.when(jnp.logical_and(c == 0, s < 10))
    def _():
        pltpu.sync_copy(acc.at[pl.ds(1000 * s, 1000)],
                        out0.at[pl.ds(1000 * s, 1000)])

    @pl.when(jnp.logical_and(c == 1, s < 10))
    def _():
        pltpu.sync_copy(acc.at[pl.ds(1000 * s, 1000)],
                        out1.at[pl.ds(1000 * s, 1000)])


def _sc_deg(dst3d, ones_h, zerosN_h):
    @functools.partial(
        pl.kernel,
        out_type=(jax.ShapeDtypeStruct((_NPAD,), _f32),) * 2,
        mesh=_mesh(),
        scratch_types=[
            pltpu.VMEM((8, _K), jnp.int32),
            pltpu.VMEM((_K,), _f32),
            pltpu.VMEM_SHARED((_NPAD,), _f32),
        ],
    )
    def k(dst_h, ones_hbm, zeros_hbm, deg0, deg1, didx, ones_v, dacc):
        c = lax.axis_index("c")
        s = lax.axis_index("s")
        wid = c * _NS + s
        pltpu.sync_copy(ones_hbm, ones_v)

        @pl.when(s == 0)
        def _():
            pltpu.sync_copy(zeros_hbm, dacc)

        plsc.subcore_barrier()

        def step(t, carry):
            blk = wid + 32 * t

            @pl.when(blk < _NBLK)
            def _():
                pltpu.sync_copy(dst_h.at[blk], didx)
                for j in range(8):
                    pltpu.sync_copy(ones_v, dacc.at[didx.at[j]], add=True)

            return carry

        lax.fori_loop(0, 10, step, 0)
        plsc.subcore_barrier()

        @pl.when(jnp.logical_and(c == 0, s == 0))
        def _():
            pltpu.sync_copy(dacc, deg0)

        @pl.when(jnp.logical_and(c == 1, s == 0))
        def _():
            pltpu.sync_copy(dacc, deg1)

    return k(dst3d, ones_h, zerosN_h)


def _sc_agg(ha, hb, src3d, dst3d, zeros_h):
    @functools.partial(
        pl.kernel,
        out_type=(jax.ShapeDtypeStruct((_N, 128), _f32),) * 2,
        mesh=_mesh(),
        scratch_types=[
            pltpu.VMEM((2, 8, _K), jnp.int32),
            pltpu.VMEM((2, 8, _K), jnp.int32),
            pltpu.VMEM((2, _K, 128), _f32),
            pltpu.VMEM_SHARED((_NPAD, 128), _f32),
            pltpu.SemaphoreType.DMA,
            pltpu.SemaphoreType.DMA,
            pltpu.SemaphoreType.DMA,
        ],
    )
    def k(ha_h, hb_h, src_h, dst_h, zeros_hbm, outa, outb,
          sidx, didx, rows, acc, semg, sems, semi):
        c = lax.axis_index("c")
        s = lax.axis_index("s")
        _zero_acc(zeros_hbm, acc, s)
        plsc.subcore_barrier()

        @pl.when(c == 0)
        def _():
            _emit_pipeline(ha_h, src_h, dst_h, sidx, didx, rows, acc,
                           semg, sems, semi, s, _NS, 20)

        @pl.when(c == 1)
        def _():
            _emit_pipeline(hb_h, src_h, dst_h, sidx, didx, rows, acc,
                           semg, sems, semi, s, _NS, 20)

        plsc.subcore_barrier()
        _readout(acc, outa, outb, c, s)

    return k(ha, hb, src3d, dst3d, zeros_h)


def _sc_agg_edges(h, src3d, dst3d, zeros_h):
    @functools.partial(
        pl.kernel,
        out_type=(jax.ShapeDtypeStruct((_N, 128), _f32),) * 2,
        mesh=_mesh(),
        scratch_types=[
            pltpu.VMEM((2, 8, _K), jnp.int32),
            pltpu.VMEM((2, 8, _K), jnp.int32),
            pltpu.VMEM((2, _K, 128), _f32),
            pltpu.VMEM_SHARED((_NPAD, 128), _f32),
            pltpu.SemaphoreType.DMA,
            pltpu.SemaphoreType.DMA,
            pltpu.SemaphoreType.DMA,
        ],
    )
    def k(h_h, src_h, dst_h, zeros_hbm, out0, out1,
          sidx, didx, rows, acc, semg, sems, semi):
        c = lax.axis_index("c")
        s = lax.axis_index("s")
        wid = c * _NS + s
        _zero_acc(zeros_hbm, acc, s)
        plsc.subcore_barrier()
        _emit_pipeline(h_h, src_h, dst_h, sidx, didx, rows, acc,
                       semg, sems, semi, wid, 32, 10)
        plsc.subcore_barrier()
        _readout(acc, out0, out1, c, s)

    return k(h, src3d, dst3d, zeros_h)


def _sc_agg_l0(xp, dp, src3d, dst3d, zeros2d, zerosN_h):
    @functools.partial(
        pl.kernel,
        out_type=(jax.ShapeDtypeStruct((_N, 128), _f32),
                  jax.ShapeDtypeStruct((_N, 128), _f32),
                  jax.ShapeDtypeStruct((_NPAD,), _f32),
                  jax.ShapeDtypeStruct((_NPAD,), _f32)),
        mesh=_mesh(),
        scratch_types=[
            pltpu.VMEM((2, 8, _K), jnp.int32),
            pltpu.VMEM((2, 8, _K), jnp.int32),
            pltpu.VMEM((2, _K, 128), _f32),
            pltpu.VMEM((2, _K), _f32),
            pltpu.VMEM_SHARED((_NPAD, 128), _f32),
            pltpu.VMEM_SHARED((_NPAD,), _f32),
            pltpu.SemaphoreType.DMA,
            pltpu.SemaphoreType.DMA,
            pltpu.SemaphoreType.DMA,
            pltpu.SemaphoreType.DMA,
            pltpu.SemaphoreType.DMA,
        ],
    )
    def k(xp_h, dp_h, src_h, dst_h, z2_h, zN_h,
          s0, s1, sd0, sd1,
          sidx, didx, rows, vals, acc, accd,
          semg, sems, semi, semg2, sems2):
        c = lax.axis_index("c")
        s = lax.axis_index("s")
        wid = c * _NS + s
        _zero_acc(z2_h, acc, s)

        @pl.when(s == 15)
        def _():
            pltpu.sync_copy(zN_h, accd)

        plsc.subcore_barrier()
        _emit_pipeline(xp_h, src_h, dst_h, sidx, didx, rows, acc,
                       semg, sems, semi, wid, 32, 10,
                       elem=(dp_h, vals, accd, semg2, sems2))
        plsc.subcore_barrier()
        _readout(acc, s0, s1, c, s)

        @pl.when(jnp.logical_and(c == 0, s == 15))
        def _():
            pltpu.sync_copy(accd, sd0)

        @pl.when(jnp.logical_and(c == 1, s == 15))
        def _():
            pltpu.sync_copy(accd, sd1)

    return k(xp, dp, src3d, dst3d, zeros2d, zerosN_h)


def _sc_pool(emb, batch3d, zeros_h):
    @functools.partial(
        pl.kernel,
        out_type=jax.ShapeDtypeStruct((_NG, 128), _f32),
        mesh=_mesh(),
        scratch_types=[
            pltpu.VMEM((200, 128), _f32),
            pltpu.VMEM((2, 100), jnp.int32),
            pltpu.VMEM((32, 128), _f32),
            pltpu.VMEM_SHARED((_NG, 128), _f32),
        ],
    )
    def k(emb_h, batch_h, zeros_hbm, g, rows, bidx, zv, gacc):
        c = lax.axis_index("c")
        s = lax.axis_index("s")

        @pl.when(c == 0)
        def _():
            pltpu.sync_copy(zeros_hbm, zv)
            pltpu.sync_copy(zv, gacc.at[pl.ds(32 * s, 32)])
            plsc.subcore_barrier()
            for j in range(4):
                chunk = s + 16 * j

                @pl.when(chunk < 50)
                def _():
                    pltpu.sync_copy(emb_h.at[pl.ds(chunk * 200, 200)], rows)
                    pltpu.sync_copy(batch_h.at[chunk], bidx)
                    for t in range(2):
                        pltpu.sync_copy(rows.at[pl.ds(t * 100, 100)],
                                        gacc.at[bidx.at[t]], add=True)

            plsc.subcore_barrier()
            pltpu.sync_copy(gacc.at[pl.ds(32 * s, 32)], g.at[pl.ds(32 * s, 32)])

    return k(emb, batch3d, zeros_h)


_R = 1000
_GRID = _N // _R


def _rspec(width):
    return pl.BlockSpec((_R, width), lambda i: (i, 0))


def _fspec(shape):
    return pl.BlockSpec(shape, lambda i: (0, 0))


def _tc_prologue(x, deg0, deg1):
    def body(x_r, d0_r, d1_r, xp_r, dp_r, dis_r):
        deg = d0_r[...] + d1_r[...]
        dis = 1.0 / jnp.sqrt(deg + 1.0)
        xp_r[...] = x_r[...] * dis
        dp_r[...] = deg * dis
        dis_r[...] = dis

    return pl.pallas_call(
        body,
        grid=(_GRID,),
        in_specs=[_rspec(128), _rspec(1), _rspec(1)],
        out_specs=(_rspec(128), _rspec(1), _rspec(1)),
        out_shape=(jax.ShapeDtypeStruct((_N, 128), _f32),
                   jax.ShapeDtypeStruct((_N, 1), _f32),
                   jax.ShapeDtypeStruct((_N, 1), _f32)),
    )(x, deg0, deg1)


_BN_SCALE = 1.0 / math.sqrt(1.0 + 1e-5)


def _tc_layer0(s0, s1, xp, sd0, sd1, dp, dis,
               w0x, w0d, b0, g0, be0, w1):
    def body(s0_r, s1_r, xp_r, sd0_r, sd1_r, dp_r, dis_r,
             w0x_r, w0d_r, b0_r, g0_r, be0_r, w1_r, ua_r, ub_r):
        dis = dis_r[...]
        aggx = (s0_r[...] + s1_r[...] + xp_r[...]) * dis
        aggd = (sd0_r[...] + sd1_r[...] + dp_r[...]) * dis
        h = (jnp.dot(aggx, w0x_r[...], precision=_HIGH,
                     preferred_element_type=_f32)
             + aggd * w0d_r[...] + b0_r[...])
        h = h * _BN_SCALE * g0_r[...] + be0_r[...]
        h = jnp.maximum(h, 0.0)
        u = jnp.dot(h, w1_r[...], precision=_HIGH,
                    preferred_element_type=_f32) * dis
        ua_r[...] = u[:, :128]
        ub_r[...] = u[:, 128:]

    return pl.pallas_call(
        body,
        grid=(_GRID,),
        in_specs=[_rspec(128), _rspec(128), _rspec(128),
                  _rspec(1), _rspec(1), _rspec(1), _rspec(1),
                  _fspec((128, 256)), _fspec((1, 256)), _fspec((1, 256)),
                  _fspec((1, 256)), _fspec((1, 256)), _fspec((256, 256))],
        out_specs=(_rspec(128), _rspec(128)),
        out_shape=(jax.ShapeDtypeStruct((_N, 128), _f32),
                   jax.ShapeDtypeStruct((_N, 128), _f32)),
    )(s0, s1, xp, sd0, sd1, dp, dis, w0x, w0d, b0, g0, be0, w1)


def _tc_mid(sa, sb, ua, ub, dis, gbn, bbn, bias, w, din, dout, split_out):
    half_in = din // 2
    half_out = dout // 2

    def body(sa_r, sb_r, ua_r, ub_r, dis_r, g_r, b_r, bias_r, w_r, *outs):
        dis = dis_r[...]
        h = jnp.concatenate(
            [sa_r[...] + ua_r[...], sb_r[...] + ub_r[...]], axis=1) * dis
        h = h + bias_r[...]
        h = h * _BN_SCALE * g_r[...] + b_r[...]
        h = jnp.maximum(h, 0.0)
        v = jnp.dot(h, w_r[...], precision=_HIGH,
                    preferred_element_type=_f32) * dis
        if split_out:
            outs[0][...] = v[:, :half_out]
            outs[1][...] = v[:, half_out:]
        else:
            outs[0][...] = v

    if split_out:
        out_specs = (_rspec(half_out), _rspec(half_out))
        out_shape = (jax.ShapeDtypeStruct((_N, half_out), _f32),
                     jax.ShapeDtypeStruct((_N, half_out), _f32))
    else:
        out_specs = _rspec(dout)
        out_shape = jax.ShapeDtypeStruct((_N, dout), _f32)
    return pl.pallas_call(
        body,
        grid=(_GRID,),
        in_specs=[_rspec(half_in), _rspec(half_in),
                  _rspec(half_in), _rspec(half_in), _rspec(1),
                  _fspec((1, din)), _fspec((1, din)), _fspec((1, din)),
                  _fspec((din, dout))],
        out_specs=out_specs,
        out_shape=out_shape,
    )(sa, sb, ua, ub, dis, gbn, bbn, bias, w)


def _tc_emb(s0, s1, u3, dis, g3, b3, bias3):
    def body(s0_r, s1_r, u3_r, dis_r, g_r, b_r, bias_r, emb_r):
        h = (s0_r[...] + s1_r[...] + u3_r[...]) * dis_r[...]
        h = h + bias_r[...]
        h = h * _BN_SCALE * g_r[...] + b_r[...]
        emb_r[...] = jnp.where(h > 0, h, 0.2 * h)

    return pl.pallas_call(
        body,
        grid=(_GRID,),
        in_specs=[_rspec(128), _rspec(128), _rspec(128), _rspec(1),
                  _fspec((1, 128)), _fspec((1, 128)), _fspec((1, 128))],
        out_specs=_rspec(128),
        out_shape=jax.ShapeDtypeStruct((_N, 128), _f32),
    )(s0, s1, u3, dis, g3, b3, bias3)


def _ln(x, g, b):
    m = jnp.mean(x, axis=1, keepdims=True)
    v = jnp.mean((x - m) ** 2, axis=1, keepdims=True)
    return (x - m) / jnp.sqrt(v + 1e-5) * g + b


def _gelu(x):
    return 0.5 * x * (1.0 + lax.erf(x * (1.0 / math.sqrt(2.0))))


def _dotT(a, b):
    return lax.dot_general(a, b, (((1,), (1,)), ((), ())),
                           precision=_HIGH, preferred_element_type=_f32)


def _mm(a, b):
    return jnp.dot(a, b, precision=_HIGH, preferred_element_type=_f32)


def _tc_popgraph(g, pp):
    def body(g_r,
             l1w, l1b, lng, lnb, l2w, l2b, temp, theta, mu, sigma,
             inw, inb, inlng, inlnb, g0w, g0b, g1w, g1b,
             c1w, c1b, c1g, c1bb, c2w, c2b, c2g, c2bb, c3w, c3b,
             logits_r, kl_r):
        gg = g_r[...]
        h = _mm(gg, l1w[...]) + l1b[...]
        h = _ln(h, lng[...], lnb[...])
        h = _gelu(h)
        latv = _mm(h, l2w[...]) + l2b[...]

        latsq = latv * latv
        n2c = jnp.sum(latsq, axis=1, keepdims=True)
        ones_r = jnp.ones((1, 64), _f32)
        n2r = lax.dot_general(ones_r, latsq, (((1,), (1,)), ((), ())),
                              precision=_HIGH, preferred_element_type=_f32)
        gram = _dotT(latv, latv)
        d2 = jnp.maximum(n2c + n2r - 2.0 * gram, 0.0)
        dist = jnp.sqrt(d2 + 1e-6)
        adj = jax.nn.sigmoid(-temp[0, 0] * dist + theta[0, 0])
        ri = lax.broadcasted_iota(jnp.int32, (_NG, _NG), 0)
        ci = lax.broadcasted_iota(jnp.int32, (_NG, _NG), 1)
        eyef = (ri == ci).astype(_f32)
        adj = adj * (1.0 - eyef) + eyef

        mask = (adj > 0.5).astype(_f32)
        A = adj * mask
        d_bar = jnp.sum(A, axis=1, keepdims=True)
        cgrid = ci.astype(_f32)
        delta = d_bar - cgrid
        S = jnp.exp(-delta * delta)
        numer = jnp.sum(S, axis=0, keepdims=True)
        q = numer / (jnp.sum(numer) + 1e-8)
        crow = lax.broadcasted_iota(jnp.int32, (1, _NG), 1).astype(_f32)
        r = jnp.exp(-(crow - mu[0, 0]) ** 2 / (2.0 * sigma[0, 0] ** 2))
        r = r / (jnp.sum(r) + 1e-8)
        kl = jnp.sum(q * jnp.log(q / (r + 1e-8) + 1e-8))
        kl = jnp.clip(kl, 0.0, 10.0)
        kl_r[...] = jnp.reshape(kl, (1, 1))

        dispc = jnp.where(d_bar > 0, 1.0 / jnp.sqrt(d_bar), 0.0)
        dispr = lax.dot_general(dispc, eyef, (((0,), (0,)), ((), ())),
                                precision=_HIGH, preferred_element_type=_f32)
        An = dispc * A * dispr

        h = _mm(gg, inw[...]) + inb[...]
        h = _ln(h, inlng[...], inlnb[...])
        h = _gelu(h)
        h = jnp.maximum(_mm(An, _mm(h, g0w[...])) + g0b[...], 0.0)
        h = jnp.maximum(_mm(An, _mm(h, g1w[...])) + g1b[...], 0.0)
        h = _mm(h, c1w[...]) + c1b[...]
        h = _ln(h, c1g[...], c1bb[...])
        h = _gelu(h)
        h = _mm(h, c2w[...]) + c2b[...]
        h = _ln(h, c2g[...], c2bb[...])
        h = _gelu(h)
        logits_r[...] = _mm(h, c3w[...]) + c3b[...]

    n = _NG
    specs = [pl.BlockSpec(a.shape, lambda i: (0, 0)) for a in pp]
    return pl.pallas_call(
        body,
        grid=(1,),
        in_specs=[pl.BlockSpec((n, 128), lambda i: (0, 0))] + specs,
        out_specs=(pl.BlockSpec((n, 10), lambda i: (0, 0)),
                   pl.BlockSpec((1, 1), lambda i: (0, 0))),
        out_shape=(jax.ShapeDtypeStruct((n, 10), _f32),
                   jax.ShapeDtypeStruct((1, 1), _f32)),
    )(g, *pp)


def kernel(x, edge_index, batch, params):
    p = params
    npad = _EPAD - _E
    padi = jnp.arange(npad, dtype=jnp.int32)
    srcp = jnp.concatenate([edge_index[0], padi % 64])
    dstp = jnp.concatenate([edge_index[1], _N + (padi % 8)])
    src3d = srcp.reshape(_NBLK, 8, _K)
    dst3d = dstp.reshape(_NBLK, 8, _K)
    batch3d = batch.reshape(50, 2, 100)

    zerosN = jnp.zeros((_NPAD,), _f32)
    ones128 = jnp.ones((_K,), _f32)
    z128 = jnp.zeros((125, 128), _f32)
    z32 = jnp.zeros((32, 128), _f32)

    def col(v):
        return v.reshape(_N, 1)

    deg0, deg1 = _sc_deg(dst3d, ones128, zerosN)
    xp, dp, dis = _tc_prologue(x, col(deg0[:_N]), col(deg1[:_N]))

    s0, s1, sd0, sd1 = _sc_agg_l0(xp, dp.reshape(_N), src3d, dst3d,
                                  z128, zerosN)

    def row(v, width):
        return v.reshape(1, width)

    u1a, u1b = _tc_layer0(
        s0, s1, xp, col(sd0[:_N]), col(sd1[:_N]), dp, dis,
        p["f1_w0"][:128], p["f1_w0"][128:129], row(p["f1_b0"], 256),
        row(p["f1_bn_g0"], 256), row(p["f1_bn_b0"], 256), p["f1_w1"])

    s1a, s1b = _sc_agg(u1a, u1b, src3d, dst3d, z128)
    u2a, u2b = _tc_mid(s1a, s1b, u1a, u1b, dis,
                       row(p["f1_bn_g1"], 256), row(p["f1_bn_b1"], 256),
                       row(p["f1_b1"], 256), p["f1_w2"], 256, 256, True)

    s2a, s2b = _sc_agg(u2a, u2b, src3d, dst3d, z128)
    u3 = _tc_mid(s2a, s2b, u2a, u2b, dis,
                 row(p["f1_bn_g2"], 256), row(p["f1_bn_b2"], 256),
                 row(p["f1_b2"], 256), p["f1_w3"], 256, 128, False)

    s3p0, s3p1 = _sc_agg_edges(u3, src3d, dst3d, z128)
    emb = _tc_emb(s3p0, s3p1, u3, dis,
                  row(p["f1_bn_g3"], 128), row(p["f1_bn_b3"], 128),
                  row(p["f1_b3"], 128))

    g = _sc_pool(emb, batch3d, z32)

    def s11(v):
        return v.reshape(1, 1)

    pp = (p["f2_l1_w"], row(p["f2_l1_b"], 64),
          row(p["f2_ln_g"], 64), row(p["f2_ln_b"], 64),
          p["f2_l2_w"], row(p["f2_l2_b"], 64),
          s11(p["f2_temp"]), s11(p["f2_theta"]),
          s11(p["f2_mu"]), s11(p["f2_sigma"]),
          p["f3_in_w"], row(p["f3_in_b"], 256),
          row(p["f3_in_ln_g"], 256), row(p["f3_in_ln_b"], 256),
          p["f3_g0_w"], row(p["f3_g0_b"], 256),
          p["f3_g1_w"], row(p["f3_g1_b"], 256),
          p["f3_c1_w"], row(p["f3_c1_b"], 512),
          row(p["f3_c1_ln_g"], 512), row(p["f3_c1_ln_b"], 512),
          p["f3_c2_w"], row(p["f3_c2_b"], 512),
          row(p["f3_c2_ln_g"], 512), row(p["f3_c2_ln_b"], 512),
          p["f3_c3_w"], row(p["f3_c3_b"], 10))

    logits, kl = _tc_popgraph(g, pp)
    return logits, jnp.reshape(kl, ())


# speedup vs baseline: 22.0829x; 1.2288x over previous
import functools
import math

import jax
import jax.numpy as jnp
from jax import lax
from jax.experimental import pallas as pl
from jax.experimental.pallas import tpu as pltpu
from jax.experimental.pallas import tpu_sc as plsc

_N = 10000
_E = 320000
_NG = 512
_NS = 16
_K = 96

_f32 = jnp.float32
_HIGH = lax.Precision.HIGHEST


def _mesh():
    return plsc.VectorSubcoreMesh(core_axis_name="c", subcore_axis_name="s")


_EAGG = _E + _N
_NBLK_AGG = -(-_EAGG // (8 * _K))
_NBLK_DEG = -(-_E // (8 * _K))
_NPAD = _N + 8


def _emit_pipeline(h_h, src_h, dst_h, sidx, didx, rows, acc,
                   semg, sems, semi, base0, stride, nt, nblk, elem=None):
    dummy = h_h.at[pl.ds(0, _K)]
    if elem is not None:
        dp_h, vals, accd, semg2, sems2 = elem
        dummy2 = dp_h.at[pl.ds(0, _K)]

    pltpu.sync_copy(src_h.at[base0], sidx.at[0])
    pltpu.sync_copy(dst_h.at[base0], didx.at[0])
    pltpu.async_copy(h_h.at[sidx.at[0, 0]], rows.at[0], semg)
    pltpu.async_copy(h_h.at[sidx.at[0, 1]], rows.at[1], semg)
    if elem is not None:
        pltpu.async_copy(dp_h.at[sidx.at[0, 0]], vals.at[0], semg2)
        pltpu.async_copy(dp_h.at[sidx.at[0, 1]], vals.at[1], semg2)

    def step(t, carry):
        blk = base0 + stride * t
        nxt = blk + stride
        slot = lax.rem(t, 2)
        nslot = lax.rem(t + 1, 2)

        @pl.when(blk < nblk)
        def _():
            @pl.when(nxt < nblk)
            def _():
                pltpu.async_copy(src_h.at[nxt], sidx.at[nslot], semi)
                pltpu.async_copy(dst_h.at[nxt], didx.at[nslot], semi)

            for j in range(8):
                b0 = lax.rem(2 * t + j, 3)
                b2 = lax.rem(2 * t + j + 2, 3)
                pltpu.make_async_copy(dummy, rows.at[b0], semg).wait()
                pltpu.async_copy(rows.at[b0], acc.at[didx.at[slot, j]],
                                 sems, add=True)
                if elem is not None:
                    pltpu.make_async_copy(dummy2, vals.at[b0], semg2).wait()
                    pltpu.async_copy(vals.at[b0], accd.at[didx.at[slot, j]],
                                     sems2, add=True)

                def _wait_prev():
                    pltpu.make_async_copy(dummy, rows.at[b2], sems).wait()
                    if elem is not None:
                        pltpu.make_async_copy(dummy2, vals.at[b2],
                                              sems2).wait()

                if j == 0:
                    @pl.when(t > 0)
                    def _():
                        _wait_prev()
                else:
                    _wait_prev()

                if j < 6:
                    pltpu.async_copy(h_h.at[sidx.at[slot, j + 2]],
                                     rows.at[b2], semg)
                    if elem is not None:
                        pltpu.async_copy(dp_h.at[sidx.at[slot, j + 2]],
                                         vals.at[b2], semg2)
                elif j == 6:
                    @pl.when(nxt < nblk)
                    def _():
                        pltpu.make_async_copy(src_h.at[0], sidx.at[nslot],
                                              semi).wait()
                        pltpu.make_async_copy(dst_h.at[0], didx.at[nslot],
                                              semi).wait()
                        pltpu.async_copy(h_h.at[sidx.at[nslot, 0]],
                                         rows.at[b2], semg)
                        if elem is not None:
                            pltpu.async_copy(dp_h.at[sidx.at[nslot, 0]],
                                             vals.at[b2], semg2)
                else:
                    @pl.when(nxt < nblk)
                    def _():
                        pltpu.async_copy(h_h.at[sidx.at[nslot, 1]],
                                         rows.at[b2], semg)
                        if elem is not None:
                            pltpu.async_copy(dp_h.at[sidx.at[nslot, 1]],
                                             vals.at[b2], semg2)

        return carry

    lax.fori_loop(0, nt, step, 0)
    pltpu.make_async_copy(dummy, rows.at[0], sems).wait()
    if elem is not None:
        pltpu.make_async_copy(dummy2, vals.at[0], sems2).wait()


def _zero_acc(zeros_hbm, acc, s):
    @pl.when(s < 10)
    def _():
        pltpu.sync_copy(zeros_hbm, acc.at[pl.ds(1000 * s, 1000)])


def _readout(acc, out0, out1, c, s):
    @pl.when(jnp.logical_and(c == 0, s < 10))
    def _():
        pltpu.sync_copy(acc.at[pl.ds(1000 * s, 1000)],
                        out0.at[pl.ds(1000 * s, 1000)])

    @pl.when(jnp.logical_and(c == 1, s < 10))
    def _():
        pltpu.sync_copy(acc.at[pl.ds(1000 * s, 1000)],
                        out1.at[pl.ds(1000 * s, 1000)])


def _sc_deg(dst3d, ones_h, zerosN_h):
    @functools.partial(
        pl.kernel,
        out_type=(jax.ShapeDtypeStruct((_NPAD,), _f32),) * 2,
        mesh=_mesh(),
        scratch_types=[
            pltpu.VMEM((8, _K), jnp.int32),
            pltpu.VMEM((_K,), _f32),
            pltpu.VMEM_SHARED((_NPAD,), _f32),
        ],
    )
    def k(dst_h, ones_hbm, zeros_hbm, deg0, deg1, didx, ones_v, dacc):
        c = lax.axis_index("c")
        s = lax.axis_index("s")
        wid = c * _NS + s
        pltpu.sync_copy(ones_hbm, ones_v)

        @pl.when(s == 0)
        def _():
            pltpu.sync_copy(zeros_hbm, dacc)

        plsc.subcore_barrier()

        def step(t, carry):
            blk = wid + 32 * t

            @pl.when(blk < _NBLK_DEG)
            def _():
                pltpu.sync_copy(dst_h.at[blk], didx)
                for j in range(8):
                    pltpu.sync_copy(ones_v, dacc.at[didx.at[j]], add=True)

            return carry

        lax.fori_loop(0, 14, step, 0)
        plsc.subcore_barrier()

        @pl.when(jnp.logical_and(c == 0, s == 0))
        def _():
            pltpu.sync_copy(dacc, deg0)

        @pl.when(jnp.logical_and(c == 1, s == 0))
        def _():
            pltpu.sync_copy(dacc, deg1)

    return k(dst3d, ones_h, zerosN_h)


def _sc_agg(ha, hb, src3d, dst3d, zeros_h):
    @functools.partial(
        pl.kernel,
        out_type=(jax.ShapeDtypeStruct((_N, 128), _f32),) * 2,
        mesh=_mesh(),
        scratch_types=[
            pltpu.VMEM((2, 8, _K), jnp.int32),
            pltpu.VMEM((2, 8, _K), jnp.int32),
            pltpu.VMEM((3, _K, 128), _f32),
            pltpu.VMEM_SHARED((_NPAD, 128), _f32),
            pltpu.SemaphoreType.DMA,
            pltpu.SemaphoreType.DMA,
            pltpu.SemaphoreType.DMA,
        ],
    )
    def k(ha_h, hb_h, src_h, dst_h, zeros_hbm, outa, outb,
          sidx, didx, rows, acc, semg, sems, semi):
        c = lax.axis_index("c")
        s = lax.axis_index("s")
        _zero_acc(zeros_hbm, acc, s)
        plsc.subcore_barrier()

        @pl.when(c == 0)
        def _():
            _emit_pipeline(ha_h, src_h, dst_h, sidx, didx, rows, acc,
                           semg, sems, semi, s, _NS, 27, _NBLK_AGG)

        @pl.when(c == 1)
        def _():
            _emit_pipeline(hb_h, src_h, dst_h, sidx, didx, rows, acc,
                           semg, sems, semi, s, _NS, 27, _NBLK_AGG)

        plsc.subcore_barrier()
        _readout(acc, outa, outb, c, s)

    return k(ha, hb, src3d, dst3d, zeros_h)


def _sc_agg_edges(h, src3d, dst3d, zeros_h):
    @functools.partial(
        pl.kernel,
        out_type=(jax.ShapeDtypeStruct((_N, 128), _f32),) * 2,
        mesh=_mesh(),
        scratch_types=[
            pltpu.VMEM((2, 8, _K), jnp.int32),
            pltpu.VMEM((2, 8, _K), jnp.int32),
            pltpu.VMEM((3, _K, 128), _f32),
            pltpu.VMEM_SHARED((_NPAD, 128), _f32),
            pltpu.SemaphoreType.DMA,
            pltpu.SemaphoreType.DMA,
            pltpu.SemaphoreType.DMA,
        ],
    )
    def k(h_h, src_h, dst_h, zeros_hbm, out0, out1,
          sidx, didx, rows, acc, semg, sems, semi):
        c = lax.axis_index("c")
        s = lax.axis_index("s")
        wid = c * _NS + s
        _zero_acc(zeros_hbm, acc, s)
        plsc.subcore_barrier()
        _emit_pipeline(h_h, src_h, dst_h, sidx, didx, rows, acc,
                       semg, sems, semi, wid, 32, 14, _NBLK_AGG)
        plsc.subcore_barrier()
        _readout(acc, out0, out1, c, s)

    return k(h, src3d, dst3d, zeros_h)


def _sc_agg_l0(xp, dp, src3d, dst3d, zeros2d, zerosN_h):
    @functools.partial(
        pl.kernel,
        out_type=(jax.ShapeDtypeStruct((_N, 128), _f32),
                  jax.ShapeDtypeStruct((_N, 128), _f32),
                  jax.ShapeDtypeStruct((_NPAD,), _f32),
                  jax.ShapeDtypeStruct((_NPAD,), _f32)),
        mesh=_mesh(),
        scratch_types=[
            pltpu.VMEM((2, 8, _K), jnp.int32),
            pltpu.VMEM((2, 8, _K), jnp.int32),
            pltpu.VMEM((3, _K, 128), _f32),
            pltpu.VMEM((3, _K), _f32),
            pltpu.VMEM_SHARED((_NPAD, 128), _f32),
            pltpu.VMEM_SHARED((_NPAD,), _f32),
            pltpu.SemaphoreType.DMA,
            pltpu.SemaphoreType.DMA,
            pltpu.SemaphoreType.DMA,
            pltpu.SemaphoreType.DMA,
            pltpu.SemaphoreType.DMA,
        ],
    )
    def k(xp_h, dp_h, src_h, dst_h, z2_h, zN_h,
          s0, s1, sd0, sd1,
          sidx, didx, rows, vals, acc, accd,
          semg, sems, semi, semg2, sems2):
        c = lax.axis_index("c")
        s = lax.axis_index("s")
        wid = c * _NS + s
        _zero_acc(z2_h, acc, s)

        @pl.when(s == 15)
        def _():
            pltpu.sync_copy(zN_h, accd)

        plsc.subcore_barrier()
        _emit_pipeline(xp_h, src_h, dst_h, sidx, didx, rows, acc,
                       semg, sems, semi, wid, 32, 14, _NBLK_AGG,
                       elem=(dp_h, vals, accd, semg2, sems2))
        plsc.subcore_barrier()
        _readout(acc, s0, s1, c, s)

        @pl.when(jnp.logical_and(c == 0, s == 15))
        def _():
            pltpu.sync_copy(accd, sd0)

        @pl.when(jnp.logical_and(c == 1, s == 15))
        def _():
            pltpu.sync_copy(accd, sd1)

    return k(xp, dp, src3d, dst3d, zeros2d, zerosN_h)


def _sc_pool(emb, batch3d, zeros_h):
    @functools.partial(
        pl.kernel,
        out_type=jax.ShapeDtypeStruct((_NG, 128), _f32),
        mesh=_mesh(),
        scratch_types=[
            pltpu.VMEM((200, 128), _f32),
            pltpu.VMEM((2, 100), jnp.int32),
            pltpu.VMEM((32, 128), _f32),
            pltpu.VMEM_SHARED((_NG, 128), _f32),
        ],
    )
    def k(emb_h, batch_h, zeros_hbm, g, rows, bidx, zv, gacc):
        c = lax.axis_index("c")
        s = lax.axis_index("s")

        @pl.when(c == 0)
        def _():
            pltpu.sync_copy(zeros_hbm, zv)
            pltpu.sync_copy(zv, gacc.at[pl.ds(32 * s, 32)])
            plsc.subcore_barrier()
            for j in range(4):
                chunk = s + 16 * j

                @pl.when(chunk < 50)
                def _():
                    pltpu.sync_copy(emb_h.at[pl.ds(chunk * 200, 200)], rows)
                    pltpu.sync_copy(batch_h.at[chunk], bidx)
                    for t in range(2):
                        pltpu.sync_copy(rows.at[pl.ds(t * 100, 100)],
                                        gacc.at[bidx.at[t]], add=True)

            plsc.subcore_barrier()
            pltpu.sync_copy(gacc.at[pl.ds(32 * s, 32)], g.at[pl.ds(32 * s, 32)])

    return k(emb, batch3d, zeros_h)


_R = 1000
_GRID = _N // _R


def _rspec(width):
    return pl.BlockSpec((_R, width), lambda i: (i, 0))


def _fspec(shape):
    return pl.BlockSpec(shape, lambda i: (0, 0))


def _tc_prologue(x, deg0, deg1):
    def body(x_r, d0_r, d1_r, xp_r, dp_r, dis_r):
        deg = d0_r[...] + d1_r[...]
        dis = 1.0 / jnp.sqrt(deg + 1.0)
        xp_r[...] = x_r[...] * dis
        dp_r[...] = deg * dis
        dis_r[...] = dis

    return pl.pallas_call(
        body,
        grid=(_GRID,),
        in_specs=[_rspec(128), _rspec(1), _rspec(1)],
        out_specs=(_rspec(128), _rspec(1), _rspec(1)),
        out_shape=(jax.ShapeDtypeStruct((_N, 128), _f32),
                   jax.ShapeDtypeStruct((_N, 1), _f32),
                   jax.ShapeDtypeStruct((_N, 1), _f32)),
    )(x, deg0, deg1)


_BN_SCALE = 1.0 / math.sqrt(1.0 + 1e-5)


def _tc_layer0(s0, s1, sd0, sd1, dis, w0x, w0d, b0, g0, be0, w1):
    def body(s0_r, s1_r, sd0_r, sd1_r, dis_r,
             w0x_r, w0d_r, b0_r, g0_r, be0_r, w1_r, ua_r, ub_r):
        dis = dis_r[...]
        aggx = (s0_r[...] + s1_r[...]) * dis
        aggd = (sd0_r[...] + sd1_r[...]) * dis
        h = (jnp.dot(aggx, w0x_r[...], precision=_HIGH,
                     preferred_element_type=_f32)
             + aggd * w0d_r[...] + b0_r[...])
        h = h * _BN_SCALE * g0_r[...] + be0_r[...]
        h = jnp.maximum(h, 0.0)
        u = jnp.dot(h, w1_r[...], precision=_HIGH,
                    preferred_element_type=_f32) * dis
        ua_r[...] = u[:, :128]
        ub_r[...] = u[:, 128:]

    return pl.pallas_call(
        body,
        grid=(_GRID,),
        in_specs=[_rspec(128), _rspec(128),
                  _rspec(1), _rspec(1), _rspec(1),
                  _fspec((128, 256)), _fspec((1, 256)), _fspec((1, 256)),
                  _fspec((1, 256)), _fspec((1, 256)), _fspec((256, 256))],
        out_specs=(_rspec(128), _rspec(128)),
        out_shape=(jax.ShapeDtypeStruct((_N, 128), _f32),
                   jax.ShapeDtypeStruct((_N, 128), _f32)),
    )(s0, s1, sd0, sd1, dis, w0x, w0d, b0, g0, be0, w1)


def _tc_mid(sa, sb, dis, gbn, bbn, bias, w, din, dout, split_out):
    half_out = dout // 2

    def body(sa_r, sb_r, dis_r, g_r, b_r, bias_r, w_r, *outs):
        dis = dis_r[...]
        h = jnp.concatenate([sa_r[...], sb_r[...]], axis=1) * dis
        h = h + bias_r[...]
        h = h * _BN_SCALE * g_r[...] + b_r[...]
        h = jnp.maximum(h, 0.0)
        v = jnp.dot(h, w_r[...], precision=_HIGH,
                    preferred_element_type=_f32) * dis
        if split_out:
            outs[0][...] = v[:, :half_out]
            outs[1][...] = v[:, half_out:]
        else:
            outs[0][...] = v

    if split_out:
        out_specs = (_rspec(half_out), _rspec(half_out))
        out_shape = (jax.ShapeDtypeStruct((_N, half_out), _f32),
                     jax.ShapeDtypeStruct((_N, half_out), _f32))
    else:
        out_specs = _rspec(dout)
        out_shape = jax.ShapeDtypeStruct((_N, dout), _f32)
    return pl.pallas_call(
        body,
        grid=(_GRID,),
        in_specs=[_rspec(din // 2), _rspec(din // 2), _rspec(1),
                  _fspec((1, din)), _fspec((1, din)), _fspec((1, din)),
                  _fspec((din, dout))],
        out_specs=out_specs,
        out_shape=out_shape,
    )(sa, sb, dis, gbn, bbn, bias, w)


def _tc_emb(s0, s1, dis, g3, b3, bias3):
    def body(s0_r, s1_r, dis_r, g_r, b_r, bias_r, emb_r):
        h = (s0_r[...] + s1_r[...]) * dis_r[...]
        h = h + bias_r[...]
        h = h * _BN_SCALE * g_r[...] + b_r[...]
        emb_r[...] = jnp.where(h > 0, h, 0.2 * h)

    return pl.pallas_call(
        body,
        grid=(_GRID,),
        in_specs=[_rspec(128), _rspec(128), _rspec(1),
                  _fspec((1, 128)), _fspec((1, 128)), _fspec((1, 128))],
        out_specs=_rspec(128),
        out_shape=jax.ShapeDtypeStruct((_N, 128), _f32),
    )(s0, s1, dis, g3, b3, bias3)


def _ln(x, g, b):
    m = jnp.mean(x, axis=1, keepdims=True)
    v = jnp.mean((x - m) ** 2, axis=1, keepdims=True)
    return (x - m) / jnp.sqrt(v + 1e-5) * g + b


def _gelu(x):
    return 0.5 * x * (1.0 + lax.erf(x * (1.0 / math.sqrt(2.0))))


def _dotT(a, b):
    return lax.dot_general(a, b, (((1,), (1,)), ((), ())),
                           precision=_HIGH, preferred_element_type=_f32)


def _mm(a, b):
    return jnp.dot(a, b, precision=_HIGH, preferred_element_type=_f32)


def _tc_popgraph(g, pp):
    def body(g_r,
             l1w, l1b, lng, lnb, l2w, l2b, temp, theta, mu, sigma,
             inw, inb, inlng, inlnb, g0w, g0b, g1w, g1b,
             c1w, c1b, c1g, c1bb, c2w, c2b, c2g, c2bb, c3w, c3b,
             logits_r, kl_r):
        gg = g_r[...]
        h = _mm(gg, l1w[...]) + l1b[...]
        h = _ln(h, lng[...], lnb[...])
        h = _gelu(h)
        latv = _mm(h, l2w[...]) + l2b[...]

        latsq = latv * latv
        n2c = jnp.sum(latsq, axis=1, keepdims=True)
        ones_r = jnp.ones((1, 64), _f32)
        n2r = lax.dot_general(ones_r, latsq, (((1,), (1,)), ((), ())),
                              precision=_HIGH, preferred_element_type=_f32)
        gram = _dotT(latv, latv)
        d2 = jnp.maximum(n2c + n2r - 2.0 * gram, 0.0)
        dist = jnp.sqrt(d2 + 1e-6)
        adj = jax.nn.sigmoid(-temp[0, 0] * dist + theta[0, 0])
        ri = lax.broadcasted_iota(jnp.int32, (_NG, _NG), 0)
        ci = lax.broadcasted_iota(jnp.int32, (_NG, _NG), 1)
        eyef = (ri == ci).astype(_f32)
        adj = adj * (1.0 - eyef) + eyef

        mask = (adj > 0.5).astype(_f32)
        A = adj * mask
        d_bar = jnp.sum(A, axis=1, keepdims=True)
        cgrid = ci.astype(_f32)
        delta = d_bar - cgrid
        S = jnp.exp(-delta * delta)
        numer = jnp.sum(S, axis=0, keepdims=True)
        q = numer / (jnp.sum(numer) + 1e-8)
        crow = lax.broadcasted_iota(jnp.int32, (1, _NG), 1).astype(_f32)
        r = jnp.exp(-(crow - mu[0, 0]) ** 2 / (2.0 * sigma[0, 0] ** 2))
        r = r / (jnp.sum(r) + 1e-8)
        kl = jnp.sum(q * jnp.log(q / (r + 1e-8) + 1e-8))
        kl = jnp.clip(kl, 0.0, 10.0)
        kl_r[...] = jnp.reshape(kl, (1, 1))

        dispc = jnp.where(d_bar > 0, 1.0 / jnp.sqrt(d_bar), 0.0)
        dispr = lax.dot_general(dispc, eyef, (((0,), (0,)), ((), ())),
                                precision=_HIGH, preferred_element_type=_f32)
        An = dispc * A * dispr

        h = _mm(gg, inw[...]) + inb[...]
        h = _ln(h, inlng[...], inlnb[...])
        h = _gelu(h)
        h = jnp.maximum(_mm(An, _mm(h, g0w[...])) + g0b[...], 0.0)
        h = jnp.maximum(_mm(An, _mm(h, g1w[...])) + g1b[...], 0.0)
        h = _mm(h, c1w[...]) + c1b[...]
        h = _ln(h, c1g[...], c1bb[...])
        h = _gelu(h)
        h = _mm(h, c2w[...]) + c2b[...]
        h = _ln(h, c2g[...], c2bb[...])
        h = _gelu(h)
        logits_r[...] = _mm(h, c3w[...]) + c3b[...]

    n = _NG
    specs = [pl.BlockSpec(a.shape, lambda i: (0, 0)) for a in pp]
    return pl.pallas_call(
        body,
        grid=(1,),
        in_specs=[pl.BlockSpec((n, 128), lambda i: (0, 0))] + specs,
        out_specs=(pl.BlockSpec((n, 10), lambda i: (0, 0)),
                   pl.BlockSpec((1, 1), lambda i: (0, 0))),
        out_shape=(jax.ShapeDtypeStruct((n, 10), _f32),
                   jax.ShapeDtypeStruct((1, 1), _f32)),
    )(g, *pp)


def kernel(x, edge_index, batch, params):
    p = params
    loop = jnp.arange(_N, dtype=jnp.int32)

    na = _NBLK_AGG * 8 * _K - _EAGG
    padi = jnp.arange(na, dtype=jnp.int32)
    srcA = jnp.concatenate([edge_index[0], loop, padi % 64])
    dstA = jnp.concatenate([edge_index[1], loop, _N + (padi % 8)])
    srcA3 = srcA.reshape(_NBLK_AGG, 8, _K)
    dstA3 = dstA.reshape(_NBLK_AGG, 8, _K)

    nd = _NBLK_DEG * 8 * _K - _E
    padd = jnp.arange(nd, dtype=jnp.int32)
    dstD = jnp.concatenate([edge_index[1], _N + (padd % 8)])
    dstD3 = dstD.reshape(_NBLK_DEG, 8, _K)

    batch3d = batch.reshape(50, 2, 100)

    zerosN = jnp.zeros((_NPAD,), _f32)
    onesK = jnp.ones((_K,), _f32)
    z1000 = jnp.zeros((1000, 128), _f32)
    z32 = jnp.zeros((32, 128), _f32)

    def col(v):
        return v.reshape(_N, 1)

    deg0, deg1 = _sc_deg(dstD3, onesK, zerosN)
    xp, dp, dis = _tc_prologue(x, col(deg0[:_N]), col(deg1[:_N]))

    s0, s1, sd0, sd1 = _sc_agg_l0(xp, dp.reshape(_N), srcA3, dstA3,
                                  z1000, zerosN)

    def row(v, width):
        return v.reshape(1, width)

    u1a, u1b = _tc_layer0(
        s0, s1, col(sd0[:_N]), col(sd1[:_N]), dis,
        p["f1_w0"][:128], p["f1_w0"][128:129], row(p["f1_b0"], 256),
        row(p["f1_bn_g0"], 256), row(p["f1_bn_b0"], 256), p["f1_w1"])

    s1a, s1b = _sc_agg(u1a, u1b, srcA3, dstA3, z1000)
    u2a, u2b = _tc_mid(s1a, s1b, dis,
                       row(p["f1_bn_g1"], 256), row(p["f1_bn_b1"], 256),
                       row(p["f1_b1"], 256), p["f1_w2"], 256, 256, True)

    s2a, s2b = _sc_agg(u2a, u2b, srcA3, dstA3, z1000)
    u3 = _tc_mid(s2a, s2b, dis,
                 row(p["f1_bn_g2"], 256), row(p["f1_bn_b2"], 256),
                 row(p["f1_b2"], 256), p["f1_w3"], 256, 128, False)

    s3p0, s3p1 = _sc_agg_edges(u3, srcA3, dstA3, z1000)
    emb = _tc_emb(s3p0, s3p1, dis,
                  row(p["f1_bn_g3"], 128), row(p["f1_bn_b3"], 128),
                  row(p["f1_b3"], 128))

    g = _sc_pool(emb, batch3d, z32)

    def s11(v):
        return v.reshape(1, 1)

    pp = (p["f2_l1_w"], row(p["f2_l1_b"], 64),
          row(p["f2_ln_g"], 64), row(p["f2_ln_b"], 64),
          p["f2_l2_w"], row(p["f2_l2_b"], 64),
          s11(p["f2_temp"]), s11(p["f2_theta"]),
          s11(p["f2_mu"]), s11(p["f2_sigma"]),
          p["f3_in_w"], row(p["f3_in_b"], 256),
          row(p["f3_in_ln_g"], 256), row(p["f3_in_ln_b"], 256),
          p["f3_g0_w"], row(p["f3_g0_b"], 256),
          p["f3_g1_w"], row(p["f3_g1_b"], 256),
          p["f3_c1_w"], row(p["f3_c1_b"], 512),
          row(p["f3_c1_ln_g"], 512), row(p["f3_c1_ln_b"], 512),
          p["f3_c2_w"], row(p["f3_c2_b"], 512),
          row(p["f3_c2_ln_g"], 512), row(p["f3_c2_ln_b"], 512),
          p["f3_c3_w"], row(p["f3_c3_b"], 10))

    logits, kl = _tc_popgraph(g, pp)
    return logits, jnp.reshape(kl, ())


# speedup vs baseline: 23.9844x; 1.0861x over previous
import functools
import math

import jax
import jax.numpy as jnp
from jax import lax
from jax.experimental import pallas as pl
from jax.experimental.pallas import tpu as pltpu
from jax.experimental.pallas import tpu_sc as plsc

_N = 10000
_E = 320000
_NG = 512
_NS = 16
_K = 96

_f32 = jnp.float32
_HIGH = lax.Precision.HIGHEST


def _mesh():
    return plsc.VectorSubcoreMesh(core_axis_name="c", subcore_axis_name="s")


_EAGG = _E + _N
_NBLK_AGG = -(-_EAGG // (8 * _K))
_NBLK_DEG = -(-_E // (8 * _K))
_NPAD = _N + 8


def _emit_pipeline(h_h, src_h, dst_h, sidx, didx, rows, acc,
                   semg, sems, semi, base0, stride, nt, nblk, elem=None):
    dummy = h_h.at[pl.ds(0, _K)]
    if elem is not None:
        dp_h, vals, accd, semg2, sems2 = elem
        dummy2 = dp_h.at[pl.ds(0, _K)]

    pltpu.sync_copy(src_h.at[base0], sidx.at[0])
    pltpu.sync_copy(dst_h.at[base0], didx.at[0])
    pltpu.async_copy(h_h.at[sidx.at[0, 0]], rows.at[0], semg)
    pltpu.async_copy(h_h.at[sidx.at[0, 1]], rows.at[1], semg)
    if elem is not None:
        pltpu.async_copy(dp_h.at[sidx.at[0, 0]], vals.at[0], semg2)
        pltpu.async_copy(dp_h.at[sidx.at[0, 1]], vals.at[1], semg2)

    def step(t, carry):
        blk = base0 + stride * t
        nxt = blk + stride
        slot = lax.rem(t, 2)
        nslot = lax.rem(t + 1, 2)

        @pl.when(blk < nblk)
        def _():
            @pl.when(nxt < nblk)
            def _():
                pltpu.async_copy(src_h.at[nxt], sidx.at[nslot], semi)
                pltpu.async_copy(dst_h.at[nxt], didx.at[nslot], semi)

            for j in range(8):
                b0 = lax.rem(2 * t + j, 3)
                b2 = lax.rem(2 * t + j + 2, 3)
                pltpu.make_async_copy(dummy, rows.at[b0], semg).wait()
                pltpu.async_copy(rows.at[b0], acc.at[didx.at[slot, j]],
                                 sems, add=True)
                if elem is not None:
                    pltpu.make_async_copy(dummy2, vals.at[b0], semg2).wait()
                    pltpu.async_copy(vals.at[b0], accd.at[didx.at[slot, j]],
                                     sems2, add=True)

                def _wait_prev():
                    pltpu.make_async_copy(dummy, rows.at[b2], sems).wait()
                    if elem is not None:
                        pltpu.make_async_copy(dummy2, vals.at[b2],
                                              sems2).wait()

                if j == 0:
                    @pl.when(t > 0)
                    def _():
                        _wait_prev()
                else:
                    _wait_prev()

                if j < 6:
                    pltpu.async_copy(h_h.at[sidx.at[slot, j + 2]],
                                     rows.at[b2], semg)
                    if elem is not None:
                        pltpu.async_copy(dp_h.at[sidx.at[slot, j + 2]],
                                         vals.at[b2], semg2)
                elif j == 6:
                    @pl.when(nxt < nblk)
                    def _():
                        pltpu.make_async_copy(src_h.at[0], sidx.at[nslot],
                                              semi).wait()
                        pltpu.make_async_copy(dst_h.at[0], didx.at[nslot],
                                              semi).wait()
                        pltpu.async_copy(h_h.at[sidx.at[nslot, 0]],
                                         rows.at[b2], semg)
                        if elem is not None:
                            pltpu.async_copy(dp_h.at[sidx.at[nslot, 0]],
                                             vals.at[b2], semg2)
                else:
                    @pl.when(nxt < nblk)
                    def _():
                        pltpu.async_copy(h_h.at[sidx.at[nslot, 1]],
                                         rows.at[b2], semg)
                        if elem is not None:
                            pltpu.async_copy(dp_h.at[sidx.at[nslot, 1]],
                                             vals.at[b2], semg2)

        return carry

    lax.fori_loop(0, nt, step, 0)
    pltpu.make_async_copy(dummy, rows.at[0], sems).wait()
    if elem is not None:
        pltpu.make_async_copy(dummy2, vals.at[0], sems2).wait()


def _zero_acc(zeros_hbm, acc, s):
    @pl.when(s < 10)
    def _():
        pltpu.sync_copy(zeros_hbm, acc.at[pl.ds(1000 * s, 1000)])


def _readout(acc, out0, out1, c, s):
    @pl.when(jnp.logical_and(c == 0, s < 10))
    def _():
        pltpu.sync_copy(acc.at[pl.ds(1000 * s, 1000)],
                        out0.at[pl.ds(1000 * s, 1000)])

    @pl.when(jnp.logical_and(c == 1, s < 10))
    def _():
        pltpu.sync_copy(acc.at[pl.ds(1000 * s, 1000)],
                        out1.at[pl.ds(1000 * s, 1000)])


def _sc_deg(dst3d, ones_h, zerosN_h):
    @functools.partial(
        pl.kernel,
        out_type=(jax.ShapeDtypeStruct((_NPAD,), _f32),) * 2,
        mesh=_mesh(),
        scratch_types=[
            pltpu.VMEM((8, _K), jnp.int32),
            pltpu.VMEM((_K,), _f32),
            pltpu.VMEM_SHARED((_NPAD,), _f32),
        ],
    )
    def k(dst_h, ones_hbm, zeros_hbm, deg0, deg1, didx, ones_v, dacc):
        c = lax.axis_index("c")
        s = lax.axis_index("s")
        wid = c * _NS + s
        pltpu.sync_copy(ones_hbm, ones_v)

        @pl.when(s == 0)
        def _():
            pltpu.sync_copy(zeros_hbm, dacc)

        plsc.subcore_barrier()

        def step(t, carry):
            blk = wid + 32 * t

            @pl.when(blk < _NBLK_DEG)
            def _():
                pltpu.sync_copy(dst_h.at[blk], didx)
                for j in range(8):
                    pltpu.sync_copy(ones_v, dacc.at[didx.at[j]], add=True)

            return carry

        lax.fori_loop(0, 14, step, 0)
        plsc.subcore_barrier()

        @pl.when(jnp.logical_and(c == 0, s == 0))
        def _():
            pltpu.sync_copy(dacc, deg0)

        @pl.when(jnp.logical_and(c == 1, s == 0))
        def _():
            pltpu.sync_copy(dacc, deg1)

    return k(dst3d, ones_h, zerosN_h)


def _sc_agg(ha, hb, src3d, dst3d, zeros_h):
    @functools.partial(
        pl.kernel,
        out_type=(jax.ShapeDtypeStruct((_N, 128), _f32),) * 2,
        mesh=_mesh(),
        scratch_types=[
            pltpu.VMEM((2, 8, _K), jnp.int32),
            pltpu.VMEM((2, 8, _K), jnp.int32),
            pltpu.VMEM((3, _K, 128), _f32),
            pltpu.VMEM_SHARED((_NPAD, 128), _f32),
            pltpu.SemaphoreType.DMA,
            pltpu.SemaphoreType.DMA,
            pltpu.SemaphoreType.DMA,
        ],
    )
    def k(ha_h, hb_h, src_h, dst_h, zeros_hbm, outa, outb,
          sidx, didx, rows, acc, semg, sems, semi):
        c = lax.axis_index("c")
        s = lax.axis_index("s")
        _zero_acc(zeros_hbm, acc, s)
        plsc.subcore_barrier()

        @pl.when(c == 0)
        def _():
            _emit_pipeline(ha_h, src_h, dst_h, sidx, didx, rows, acc,
                           semg, sems, semi, s, _NS, 27, _NBLK_AGG)

        @pl.when(c == 1)
        def _():
            _emit_pipeline(hb_h, src_h, dst_h, sidx, didx, rows, acc,
                           semg, sems, semi, s, _NS, 27, _NBLK_AGG)

        plsc.subcore_barrier()
        _readout(acc, outa, outb, c, s)

    return k(ha, hb, src3d, dst3d, zeros_h)


def _sc_agg_edges(h, src3d, dst3d, zeros_h):
    @functools.partial(
        pl.kernel,
        out_type=(jax.ShapeDtypeStruct((_N, 128), _f32),) * 2,
        mesh=_mesh(),
        scratch_types=[
            pltpu.VMEM((2, 8, _K), jnp.int32),
            pltpu.VMEM((2, 8, _K), jnp.int32),
            pltpu.VMEM((3, _K, 128), _f32),
            pltpu.VMEM_SHARED((_NPAD, 128), _f32),
            pltpu.SemaphoreType.DMA,
            pltpu.SemaphoreType.DMA,
            pltpu.SemaphoreType.DMA,
        ],
    )
    def k(h_h, src_h, dst_h, zeros_hbm, out0, out1,
          sidx, didx, rows, acc, semg, sems, semi):
        c = lax.axis_index("c")
        s = lax.axis_index("s")
        wid = c * _NS + s
        _zero_acc(zeros_hbm, acc, s)
        plsc.subcore_barrier()
        _emit_pipeline(h_h, src_h, dst_h, sidx, didx, rows, acc,
                       semg, sems, semi, wid, 32, 14, _NBLK_AGG)
        plsc.subcore_barrier()
        _readout(acc, out0, out1, c, s)

    return k(h, src3d, dst3d, zeros_h)


def _sc_agg_l0(xp, dp, src3d, dst3d, zeros2d, zerosN_h):
    @functools.partial(
        pl.kernel,
        out_type=(jax.ShapeDtypeStruct((_N, 128), _f32),
                  jax.ShapeDtypeStruct((_N, 128), _f32),
                  jax.ShapeDtypeStruct((_NPAD,), _f32),
                  jax.ShapeDtypeStruct((_NPAD,), _f32)),
        mesh=_mesh(),
        scratch_types=[
            pltpu.VMEM((2, 8, _K), jnp.int32),
            pltpu.VMEM((2, 8, _K), jnp.int32),
            pltpu.VMEM((3, _K, 128), _f32),
            pltpu.VMEM((3, _K), _f32),
            pltpu.VMEM_SHARED((_NPAD, 128), _f32),
            pltpu.VMEM_SHARED((_NPAD,), _f32),
            pltpu.SemaphoreType.DMA,
            pltpu.SemaphoreType.DMA,
            pltpu.SemaphoreType.DMA,
            pltpu.SemaphoreType.DMA,
            pltpu.SemaphoreType.DMA,
        ],
    )
    def k(xp_h, dp_h, src_h, dst_h, z2_h, zN_h,
          s0, s1, sd0, sd1,
          sidx, didx, rows, vals, acc, accd,
          semg, sems, semi, semg2, sems2):
        c = lax.axis_index("c")
        s = lax.axis_index("s")
        wid = c * _NS + s
        _zero_acc(z2_h, acc, s)

        @pl.when(s == 15)
        def _():
            pltpu.sync_copy(zN_h, accd)

        plsc.subcore_barrier()
        _emit_pipeline(xp_h, src_h, dst_h, sidx, didx, rows, acc,
                       semg, sems, semi, wid, 32, 14, _NBLK_AGG,
                       elem=(dp_h, vals, accd, semg2, sems2))
        plsc.subcore_barrier()
        _readout(acc, s0, s1, c, s)

        @pl.when(jnp.logical_and(c == 0, s == 15))
        def _():
            pltpu.sync_copy(accd, sd0)

        @pl.when(jnp.logical_and(c == 1, s == 15))
        def _():
            pltpu.sync_copy(accd, sd1)

    return k(xp, dp, src3d, dst3d, zeros2d, zerosN_h)


def _sc_pool(emb, batch3d, zeros_h):
    @functools.partial(
        pl.kernel,
        out_type=(jax.ShapeDtypeStruct((_NG, 128), _f32),) * 2,
        mesh=_mesh(),
        scratch_types=[
            pltpu.VMEM((200, 128), _f32),
            pltpu.VMEM((2, 100), jnp.int32),
            pltpu.VMEM((32, 128), _f32),
            pltpu.VMEM_SHARED((_NG, 128), _f32),
        ],
    )
    def k(emb_h, batch_h, zeros_hbm, g0, g1, rows, bidx, zv, gacc):
        c = lax.axis_index("c")
        s = lax.axis_index("s")
        wid = c * _NS + s
        pltpu.sync_copy(zeros_hbm, zv)
        pltpu.sync_copy(zv, gacc.at[pl.ds(32 * s, 32)])
        plsc.subcore_barrier()
        for j in range(2):
            chunk = wid + 32 * j

            @pl.when(chunk < 50)
            def _():
                pltpu.sync_copy(emb_h.at[pl.ds(chunk * 200, 200)], rows)
                pltpu.sync_copy(batch_h.at[chunk], bidx)
                for t in range(2):
                    pltpu.sync_copy(rows.at[pl.ds(t * 100, 100)],
                                    gacc.at[bidx.at[t]], add=True)

        plsc.subcore_barrier()

        @pl.when(c == 0)
        def _():
            pltpu.sync_copy(gacc.at[pl.ds(32 * s, 32)],
                            g0.at[pl.ds(32 * s, 32)])

        @pl.when(c == 1)
        def _():
            pltpu.sync_copy(gacc.at[pl.ds(32 * s, 32)],
                            g1.at[pl.ds(32 * s, 32)])

    return k(emb, batch3d, zeros_h)


_R = 1000
_GRID = _N // _R


def _rspec(width):
    return pl.BlockSpec((_R, width), lambda i: (i, 0))


def _fspec(shape):
    return pl.BlockSpec(shape, lambda i: (0, 0))


def _tc_prologue(x, deg0, deg1):
    def body(x_r, d0_r, d1_r, xp_r, dp_r, dis_r):
        deg = d0_r[...] + d1_r[...]
        dis = 1.0 / jnp.sqrt(deg + 1.0)
        xp_r[...] = x_r[...] * dis
        dp_r[...] = deg * dis
        dis_r[...] = dis

    return pl.pallas_call(
        body,
        grid=(_GRID,),
        in_specs=[_rspec(128), _rspec(1), _rspec(1)],
        out_specs=(_rspec(128), _rspec(1), _rspec(1)),
        out_shape=(jax.ShapeDtypeStruct((_N, 128), _f32),
                   jax.ShapeDtypeStruct((_N, 1), _f32),
                   jax.ShapeDtypeStruct((_N, 1), _f32)),
    )(x, deg0, deg1)


_BN_SCALE = 1.0 / math.sqrt(1.0 + 1e-5)


def _tc_layer0(s0, s1, sd0, sd1, dis, w0x, w0d, b0, g0, be0, w1):
    def body(s0_r, s1_r, sd0_r, sd1_r, dis_r,
             w0x_r, w0d_r, b0_r, g0_r, be0_r, w1_r, ua_r, ub_r):
        dis = dis_r[...]
        aggx = (s0_r[...] + s1_r[...]) * dis
        aggd = (sd0_r[...] + sd1_r[...]) * dis
        h = (jnp.dot(aggx, w0x_r[...],
                     preferred_element_type=_f32)
             + aggd * w0d_r[...] + b0_r[...])
        h = h * _BN_SCALE * g0_r[...] + be0_r[...]
        h = jnp.maximum(h, 0.0)
        u = jnp.dot(h, w1_r[...],
                    preferred_element_type=_f32) * dis
        ua_r[...] = u[:, :128]
        ub_r[...] = u[:, 128:]

    return pl.pallas_call(
        body,
        grid=(_GRID,),
        in_specs=[_rspec(128), _rspec(128),
                  _rspec(1), _rspec(1), _rspec(1),
                  _fspec((128, 256)), _fspec((1, 256)), _fspec((1, 256)),
                  _fspec((1, 256)), _fspec((1, 256)), _fspec((256, 256))],
        out_specs=(_rspec(128), _rspec(128)),
        out_shape=(jax.ShapeDtypeStruct((_N, 128), _f32),
                   jax.ShapeDtypeStruct((_N, 128), _f32)),
    )(s0, s1, sd0, sd1, dis, w0x, w0d, b0, g0, be0, w1)


def _tc_mid(sa, sb, dis, gbn, bbn, bias, w, din, dout, split_out):
    half_out = dout // 2

    def body(sa_r, sb_r, dis_r, g_r, b_r, bias_r, w_r, *outs):
        dis = dis_r[...]
        h = jnp.concatenate([sa_r[...], sb_r[...]], axis=1) * dis
        h = h + bias_r[...]
        h = h * _BN_SCALE * g_r[...] + b_r[...]
        h = jnp.maximum(h, 0.0)
        v = jnp.dot(h, w_r[...],
                    preferred_element_type=_f32) * dis
        if split_out:
            outs[0][...] = v[:, :half_out]
            outs[1][...] = v[:, half_out:]
        else:
            outs[0][...] = v

    if split_out:
        out_specs = (_rspec(half_out), _rspec(half_out))
        out_shape = (jax.ShapeDtypeStruct((_N, half_out), _f32),
                     jax.ShapeDtypeStruct((_N, half_out), _f32))
    else:
        out_specs = _rspec(dout)
        out_shape = jax.ShapeDtypeStruct((_N, dout), _f32)
    return pl.pallas_call(
        body,
        grid=(_GRID,),
        in_specs=[_rspec(din // 2), _rspec(din // 2), _rspec(1),
                  _fspec((1, din)), _fspec((1, din)), _fspec((1, din)),
                  _fspec((din, dout))],
        out_specs=out_specs,
        out_shape=out_shape,
    )(sa, sb, dis, gbn, bbn, bias, w)


def _tc_emb(s0, s1, dis, g3, b3, bias3):
    def body(s0_r, s1_r, dis_r, g_r, b_r, bias_r, emb_r):
        h = (s0_r[...] + s1_r[...]) * dis_r[...]
        h = h + bias_r[...]
        h = h * _BN_SCALE * g_r[...] + b_r[...]
        emb_r[...] = jnp.where(h > 0, h, 0.2 * h)

    return pl.pallas_call(
        body,
        grid=(_GRID,),
        in_specs=[_rspec(128), _rspec(128), _rspec(1),
                  _fspec((1, 128)), _fspec((1, 128)), _fspec((1, 128))],
        out_specs=_rspec(128),
        out_shape=jax.ShapeDtypeStruct((_N, 128), _f32),
    )(s0, s1, dis, g3, b3, bias3)


def _ln(x, g, b):
    m = jnp.mean(x, axis=1, keepdims=True)
    v = jnp.mean((x - m) ** 2, axis=1, keepdims=True)
    return (x - m) / jnp.sqrt(v + 1e-5) * g + b


def _gelu(x):
    return 0.5 * x * (1.0 + lax.erf(x * (1.0 / math.sqrt(2.0))))


def _dotT(a, b):
    return lax.dot_general(a, b, (((1,), (1,)), ((), ())),
                           preferred_element_type=_f32)


def _mm(a, b):
    return jnp.dot(a, b, preferred_element_type=_f32)


def _tc_popgraph(g0, g1, pp):
    def body(g0_r, g1_r,
             l1w, l1b, lng, lnb, l2w, l2b, temp, theta, mu, sigma,
             inw, inb, inlng, inlnb, g0w, g0b, g1w, g1b,
             c1w, c1b, c1g, c1bb, c2w, c2b, c2g, c2bb, c3w, c3b,
             logits_r, kl_r):
        gg = g0_r[...] + g1_r[...]
        h = _mm(gg, l1w[...]) + l1b[...]
        h = _ln(h, lng[...], lnb[...])
        h = _gelu(h)
        latv = _mm(h, l2w[...]) + l2b[...]

        latsq = latv * latv
        n2c = jnp.sum(latsq, axis=1, keepdims=True)
        ones_r = jnp.ones((1, 64), _f32)
        n2r = lax.dot_general(ones_r, latsq, (((1,), (1,)), ((), ())),
                              preferred_element_type=_f32)
        gram = _dotT(latv, latv)
        d2 = jnp.maximum(n2c + n2r - 2.0 * gram, 0.0)
        dist = jnp.sqrt(d2 + 1e-6)
        adj = jax.nn.sigmoid(-temp[0, 0] * dist + theta[0, 0])
        ri = lax.broadcasted_iota(jnp.int32, (_NG, _NG), 0)
        ci = lax.broadcasted_iota(jnp.int32, (_NG, _NG), 1)
        eyef = (ri == ci).astype(_f32)
        adj = adj * (1.0 - eyef) + eyef

        mask = (adj > 0.5).astype(_f32)
        A = adj * mask
        d_bar = jnp.sum(A, axis=1, keepdims=True)
        cgrid = ci.astype(_f32)
        delta = d_bar - cgrid
        S = jnp.exp(-delta * delta)
        numer = jnp.sum(S, axis=0, keepdims=True)
        q = numer / (jnp.sum(numer) + 1e-8)
        crow = lax.broadcasted_iota(jnp.int32, (1, _NG), 1).astype(_f32)
        r = jnp.exp(-(crow - mu[0, 0]) ** 2 / (2.0 * sigma[0, 0] ** 2))
        r = r / (jnp.sum(r) + 1e-8)
        kl = jnp.sum(q * jnp.log(q / (r + 1e-8) + 1e-8))
        kl = jnp.clip(kl, 0.0, 10.0)
        kl_r[...] = jnp.reshape(kl, (1, 1))

        dispc = jnp.where(d_bar > 0, 1.0 / jnp.sqrt(d_bar), 0.0)
        dispr = lax.dot_general(dispc, eyef, (((0,), (0,)), ((), ())),
                                preferred_element_type=_f32)
        An = dispc * A * dispr

        h = _mm(gg, inw[...]) + inb[...]
        h = _ln(h, inlng[...], inlnb[...])
        h = _gelu(h)
        h = jnp.maximum(_mm(An, _mm(h, g0w[...])) + g0b[...], 0.0)
        h = jnp.maximum(_mm(An, _mm(h, g1w[...])) + g1b[...], 0.0)
        h = _mm(h, c1w[...]) + c1b[...]
        h = _ln(h, c1g[...], c1bb[...])
        h = _gelu(h)
        h = _mm(h, c2w[...]) + c2b[...]
        h = _ln(h, c2g[...], c2bb[...])
        h = _gelu(h)
        logits_r[...] = _mm(h, c3w[...]) + c3b[...]

    n = _NG
    specs = [pl.BlockSpec(a.shape, lambda i: (0, 0)) for a in pp]
    return pl.pallas_call(
        body,
        grid=(1,),
        in_specs=[pl.BlockSpec((n, 128), lambda i: (0, 0)),
                  pl.BlockSpec((n, 128), lambda i: (0, 0))] + specs,
        out_specs=(pl.BlockSpec((n, 10), lambda i: (0, 0)),
                   pl.BlockSpec((1, 1), lambda i: (0, 0))),
        out_shape=(jax.ShapeDtypeStruct((n, 10), _f32),
                   jax.ShapeDtypeStruct((1, 1), _f32)),
    )(g0, g1, *pp)


def kernel(x, edge_index, batch, params):
    p = params
    loop = jnp.arange(_N, dtype=jnp.int32)

    na = _NBLK_AGG * 8 * _K - _EAGG
    padi = jnp.arange(na, dtype=jnp.int32)
    srcA = jnp.concatenate([edge_index[0], loop, padi % 64])
    dstA = jnp.concatenate([edge_index[1], loop, _N + (padi % 8)])
    srcA3 = srcA.reshape(_NBLK_AGG, 8, _K)
    dstA3 = dstA.reshape(_NBLK_AGG, 8, _K)

    nd = _NBLK_DEG * 8 * _K - _E
    padd = jnp.arange(nd, dtype=jnp.int32)
    dstD = jnp.concatenate([edge_index[1], _N + (padd % 8)])
    dstD3 = dstD.reshape(_NBLK_DEG, 8, _K)

    batch3d = batch.reshape(50, 2, 100)

    zerosN = jnp.zeros((_NPAD,), _f32)
    onesK = jnp.ones((_K,), _f32)
    z1000 = jnp.zeros((1000, 128), _f32)
    z32 = jnp.zeros((32, 128), _f32)

    def col(v):
        return v.reshape(_N, 1)

    deg0, deg1 = _sc_deg(dstD3, onesK, zerosN)
    xp, dp, dis = _tc_prologue(x, col(deg0[:_N]), col(deg1[:_N]))

    s0, s1, sd0, sd1 = _sc_agg_l0(xp, dp.reshape(_N), srcA3, dstA3,
                                  z1000, zerosN)

    def row(v, width):
        return v.reshape(1, width)

    u1a, u1b = _tc_layer0(
        s0, s1, col(sd0[:_N]), col(sd1[:_N]), dis,
        p["f1_w0"][:128], p["f1_w0"][128:129], row(p["f1_b0"], 256),
        row(p["f1_bn_g0"], 256), row(p["f1_bn_b0"], 256), p["f1_w1"])

    s1a, s1b = _sc_agg(u1a, u1b, srcA3, dstA3, z1000)
    u2a, u2b = _tc_mid(s1a, s1b, dis,
                       row(p["f1_bn_g1"], 256), row(p["f1_bn_b1"], 256),
                       row(p["f1_b1"], 256), p["f1_w2"], 256, 256, True)

    s2a, s2b = _sc_agg(u2a, u2b, srcA3, dstA3, z1000)
    u3 = _tc_mid(s2a, s2b, dis,
                 row(p["f1_bn_g2"], 256), row(p["f1_bn_b2"], 256),
                 row(p["f1_b2"], 256), p["f1_w3"], 256, 128, False)

    s3p0, s3p1 = _sc_agg_edges(u3, srcA3, dstA3, z1000)
    emb = _tc_emb(s3p0, s3p1, dis,
                  row(p["f1_bn_g3"], 128), row(p["f1_bn_b3"], 128),
                  row(p["f1_b3"], 128))

    g0, g1 = _sc_pool(emb, batch3d, z32)

    def s11(v):
        return v.reshape(1, 1)

    pp = (p["f2_l1_w"], row(p["f2_l1_b"], 64),
          row(p["f2_ln_g"], 64), row(p["f2_ln_b"], 64),
          p["f2_l2_w"], row(p["f2_l2_b"], 64),
          s11(p["f2_temp"]), s11(p["f2_theta"]),
          s11(p["f2_mu"]), s11(p["f2_sigma"]),
          p["f3_in_w"], row(p["f3_in_b"], 256),
          row(p["f3_in_ln_g"], 256), row(p["f3_in_ln_b"], 256),
          p["f3_g0_w"], row(p["f3_g0_b"], 256),
          p["f3_g1_w"], row(p["f3_g1_b"], 256),
          p["f3_c1_w"], row(p["f3_c1_b"], 512),
          row(p["f3_c1_ln_g"], 512), row(p["f3_c1_ln_b"], 512),
          p["f3_c2_w"], row(p["f3_c2_b"], 512),
          row(p["f3_c2_ln_g"], 512), row(p["f3_c2_ln_b"], 512),
          p["f3_c3_w"], row(p["f3_c3_b"], 10))

    logits, kl = _tc_popgraph(g0, g1, pp)
    return logits, jnp.reshape(kl, ())
